# baseline (speedup 1.0000x reference)
import functools

import jax
import jax.numpy as jnp
from jax import lax
from jax.experimental import pallas as pl
from jax.experimental.pallas import tpu as pltpu
from jax.experimental.pallas import tpu_sc as plsc

N = 10000
E = 320000
D_IN = 128
D_HID = 256
D_OUT = 128
DCOL = 128

NC = 2
NS = 16
NW = NC * NS
EPW = E // NW
C = 80
NCHUNK = EPW // C

DRAIN_ROWS = 624

_MESH = plsc.VectorSubcoreMesh(
    core_axis_name="c", subcore_axis_name="s", num_cores=NC, num_subcores=NS
)


NPAD = 10240


def _deg_body(dst_hbm, ew_hbm, zeros_hbm, out0_hbm, out1_hbm,
              acc_sh, dst_v0, ew_v0, ssem0, dst_v1, ew_v1, ssem1):
    c = lax.axis_index("c")
    s = lax.axis_index("s")
    wid = c * NS + s

    @pl.when(s == 0)
    def _():
        pltpu.sync_copy(zeros_hbm, acc_sh)

    plsc.subcore_barrier()

    base = wid * EPW
    bufs = ((dst_v0, ew_v0, ssem0), (dst_v1, ew_v1, ssem1))

    def _wait_scatter(buf):
        dst_v, ew_v, ssem = buf
        pltpu.make_async_copy(ew_v, acc_sh.at[dst_v], ssem).wait()

    def _load(k, buf):
        dst_v, ew_v, _ = buf
        off = base + k * C
        pltpu.sync_copy(dst_hbm.at[pl.ds(off, C)], dst_v)
        pltpu.sync_copy(ew_hbm.at[pl.ds(off, C)], ew_v)

    def _step(k, cur, nxt):
        dst_v, ew_v, ssem = cur

        @pl.when(k + 1 < NCHUNK)
        def _():
            @pl.when(k >= 1)
            def _():
                _wait_scatter(nxt)

            _load(k + 1, nxt)

        pltpu.async_copy(ew_v, acc_sh.at[dst_v], ssem, add=True)

    _load(0, bufs[0])

    @pl.loop(0, NCHUNK)
    def _(k):
        @pl.when(k % 2 == 0)
        def _():
            _step(k, bufs[0], bufs[1])

        @pl.when(k % 2 == 1)
        def _():
            _step(k, bufs[1], bufs[0])

    _wait_scatter(bufs[1])
    _wait_scatter(bufs[0])

    plsc.subcore_barrier()

    @pl.when((s == 0) & (c == 0))
    def _():
        pltpu.sync_copy(acc_sh, out0_hbm)

    @pl.when((s == 0) & (c == 1))
    def _():
        pltpu.sync_copy(acc_sh, out1_hbm)


def _deg_partials(dst, ew, zeros_pad):
    return pl.kernel(
        _deg_body,
        out_type=(jax.ShapeDtypeStruct((NPAD,), jnp.float32),
                  jax.ShapeDtypeStruct((NPAD,), jnp.float32)),
        mesh=_MESH,
        scratch_types=[
            pltpu.VMEM_SHARED((NPAD,), jnp.float32),
            pltpu.VMEM((C,), jnp.int32),
            pltpu.VMEM((C,), jnp.float32),
            pltpu.SemaphoreType.DMA,
            pltpu.VMEM((C,), jnp.int32),
            pltpu.VMEM((C,), jnp.float32),
            pltpu.SemaphoreType.DMA,
        ],
    )(dst, ew, zeros_pad)


def _spmm_body(g_hbm, src_hbm, dst_hbm, ew_hbm, zeros_hbm, out_hbm,
               acc_sh,
               src_v0, dst_v0, ew_v0, rows_v0, gsem0, ssem0,
               src_v1, dst_v1, ew_v1, rows_v1, gsem1, ssem1):
    c = lax.axis_index("c")
    s = lax.axis_index("s")
    wid = c * NS + s

    @pl.when(s == 0)
    def _():
        pltpu.sync_copy(zeros_hbm, acc_sh)

    plsc.subcore_barrier()

    base = wid * EPW
    bufs = ((src_v0, dst_v0, ew_v0, rows_v0, gsem0, ssem0),
            (src_v1, dst_v1, ew_v1, rows_v1, gsem1, ssem1))

    def _wait_scatter(buf):
        _, dst_v, _, rows_v, _, ssem = buf
        pltpu.make_async_copy(rows_v, acc_sh.at[dst_v], ssem).wait()

    def _load_and_start(k, buf):
        src_v, dst_v, ew_v, rows_v, gsem, _ = buf
        off = base + k * C
        pltpu.sync_copy(src_hbm.at[pl.ds(off, C)], src_v)
        pltpu.sync_copy(dst_hbm.at[pl.ds(off, C)], dst_v)
        pltpu.sync_copy(ew_hbm.at[pl.ds(off, C)], ew_v)
        pltpu.async_copy(g_hbm.at[src_v], rows_v, gsem)

    def _step(k, cur, nxt):
        src_v, dst_v, ew_v, rows_v, gsem, ssem = cur
        pltpu.make_async_copy(g_hbm.at[src_v], rows_v, gsem).wait()

        @pl.when(k + 1 < NCHUNK)
        def _():
            @pl.when(k >= 1)
            def _():
                _wait_scatter(nxt)

            _load_and_start(k + 1, nxt)

        @pl.loop(0, C // 16)
        def _(gidx):
            wvec = ew_v[pl.ds(gidx * 16, 16)]
            for lane in range(16):
                w = jnp.full((16,), wvec[lane])
                r = gidx * 16 + lane
                for j in range(DCOL // 16):
                    sl = pl.ds(j * 16, 16)
                    rows_v[r, sl] = rows_v[r, sl] * w

        pltpu.async_copy(rows_v, acc_sh.at[dst_v], ssem, add=True)

    _load_and_start(0, bufs[0])

    @pl.loop(0, NCHUNK)
    def _(k):
        @pl.when(k % 2 == 0)
        def _():
            _step(k, bufs[0], bufs[1])

        @pl.when(k % 2 == 1)
        def _():
            _step(k, bufs[1], bufs[0])

    _wait_scatter(bufs[1])
    _wait_scatter(bufs[0])

    plsc.subcore_barrier()

    row0 = s * DRAIN_ROWS

    @pl.when(s < NS - 1)
    def _():
        pltpu.sync_copy(acc_sh.at[pl.ds(row0, DRAIN_ROWS)],
                        out_hbm.at[c, pl.ds(row0, DRAIN_ROWS)])

    @pl.when(s == NS - 1)
    def _():
        last = (NS - 1) * DRAIN_ROWS
        pltpu.sync_copy(acc_sh.at[pl.ds(last, N - last)],
                        out_hbm.at[c, pl.ds(last, N - last)])


def _spmm_partials(g, src, dst, ew, zeros_nd):
    return pl.kernel(
        _spmm_body,
        out_type=jax.ShapeDtypeStruct((NC, N, DCOL), jnp.float32),
        mesh=_MESH,
        scratch_types=[
            pltpu.VMEM_SHARED((N, DCOL), jnp.float32),
            pltpu.VMEM((C,), jnp.int32),
            pltpu.VMEM((C,), jnp.int32),
            pltpu.VMEM((C,), jnp.float32),
            pltpu.VMEM((C, DCOL), jnp.float32),
            pltpu.SemaphoreType.DMA,
            pltpu.SemaphoreType.DMA,
            pltpu.VMEM((C,), jnp.int32),
            pltpu.VMEM((C,), jnp.int32),
            pltpu.VMEM((C,), jnp.float32),
            pltpu.VMEM((C, DCOL), jnp.float32),
            pltpu.SemaphoreType.DMA,
            pltpu.SemaphoreType.DMA,
        ],
    )(g, src, dst, ew, zeros_nd)


BLK = 1000


def _mm1_body(degp_ref, x_ref, w_ref, dis_ref, ga_ref, gb_ref):
    deg = degp_ref[0] + degp_ref[1] + 1.0
    dis = lax.rsqrt(deg)
    h = jnp.dot(x_ref[...], w_ref[...],
                preferred_element_type=jnp.float32,
                precision=lax.Precision.HIGHEST)
    g = h * dis
    dis_ref[...] = dis
    ga_ref[...] = g[:, :DCOL]
    gb_ref[...] = g[:, DCOL:]


def _mm1(degp, x, W1):
    return pl.pallas_call(
        _mm1_body,
        grid=(N // BLK,),
        in_specs=[
            pl.BlockSpec((NC, BLK, 1), lambda i: (0, i, 0)),
            pl.BlockSpec((BLK, D_IN), lambda i: (i, 0)),
            pl.BlockSpec((D_IN, D_HID), lambda i: (0, 0)),
        ],
        out_specs=[
            pl.BlockSpec((BLK, 1), lambda i: (i, 0)),
            pl.BlockSpec((BLK, DCOL), lambda i: (i, 0)),
            pl.BlockSpec((BLK, DCOL), lambda i: (i, 0)),
        ],
        out_shape=[
            jax.ShapeDtypeStruct((N, 1), jnp.float32),
            jax.ShapeDtypeStruct((N, DCOL), jnp.float32),
            jax.ShapeDtypeStruct((N, DCOL), jnp.float32),
        ],
    )(degp, x, W1)


def _mid_body(s1a_ref, s1b_ref, ga_ref, gb_ref, dis_ref, b1_ref, w2_ref,
              g2_ref):
    dis = dis_ref[...]
    b1 = b1_ref[...]
    za = dis * (s1a_ref[0] + s1a_ref[1] + ga_ref[...]) + b1[:, :DCOL]
    zb = dis * (s1b_ref[0] + s1b_ref[1] + gb_ref[...]) + b1[:, DCOL:]
    z = jnp.maximum(jnp.concatenate([za, zb], axis=1), 0.0)
    h2 = jnp.dot(z, w2_ref[...],
                 preferred_element_type=jnp.float32,
                 precision=lax.Precision.HIGHEST)
    g2_ref[...] = h2 * dis


def _mid(s1a, s1b, ga, gb, dis, b1, W2):
    return pl.pallas_call(
        _mid_body,
        grid=(N // BLK,),
        in_specs=[
            pl.BlockSpec((NC, BLK, DCOL), lambda i: (0, i, 0)),
            pl.BlockSpec((NC, BLK, DCOL), lambda i: (0, i, 0)),
            pl.BlockSpec((BLK, DCOL), lambda i: (i, 0)),
            pl.BlockSpec((BLK, DCOL), lambda i: (i, 0)),
            pl.BlockSpec((BLK, 1), lambda i: (i, 0)),
            pl.BlockSpec((1, D_HID), lambda i: (0, 0)),
            pl.BlockSpec((D_HID, D_OUT), lambda i: (0, 0)),
        ],
        out_specs=pl.BlockSpec((BLK, D_OUT), lambda i: (i, 0)),
        out_shape=jax.ShapeDtypeStruct((N, D_OUT), jnp.float32),
    )(s1a, s1b, ga, gb, dis, b1, W2)


def _fin_body(s2_ref, g2_ref, dis_ref, b2_ref, out_ref):
    dis = dis_ref[...]
    out_ref[...] = (dis * (s2_ref[0] + s2_ref[1] + g2_ref[...])
                    + b2_ref[...])


def _fin(s2, g2, dis, b2):
    return pl.pallas_call(
        _fin_body,
        grid=(N // BLK,),
        in_specs=[
            pl.BlockSpec((NC, BLK, D_OUT), lambda i: (0, i, 0)),
            pl.BlockSpec((BLK, D_OUT), lambda i: (i, 0)),
            pl.BlockSpec((BLK, 1), lambda i: (i, 0)),
            pl.BlockSpec((1, D_OUT), lambda i: (0, 0)),
        ],
        out_specs=pl.BlockSpec((BLK, D_OUT), lambda i: (i, 0)),
        out_shape=jax.ShapeDtypeStruct((N, D_OUT), jnp.float32),
    )(s2, g2, dis, b2)


def kernel(x, edge_index, edge_weight, W1, b1, W2, b2):
    ei = edge_index.astype(jnp.int32)
    src = ei[0]
    dst = ei[1]
    ew = edge_weight.astype(jnp.float32)
    zeros_nd = jnp.zeros((N, DCOL), jnp.float32)

    degp0, degp1 = _deg_partials(dst, ew, jnp.zeros((NPAD,), jnp.float32))
    degp = jnp.stack([degp0[:N], degp1[:N]]).reshape(NC, N, 1)
    dis, g1a, g1b = _mm1(degp, x, W1)
    s1a = _spmm_partials(g1a, src, dst, ew, zeros_nd)
    s1b = _spmm_partials(g1b, src, dst, ew, zeros_nd)
    g2 = _mid(s1a, s1b, g1a, g1b, dis, b1.reshape(1, D_HID), W2)
    s2 = _spmm_partials(g2, src, dst, ew, zeros_nd)
    return _fin(s2, g2, dis, b2.reshape(1, D_OUT))


# speedup vs baseline: 1.3645x; 1.3645x over previous
import functools

import jax
import jax.numpy as jnp
from jax import lax
from jax.experimental import pallas as pl
from jax.experimental.pallas import tpu as pltpu
from jax.experimental.pallas import tpu_sc as plsc

N = 10000
E = 320000
D_IN = 128
D_HID = 256
D_OUT = 128
DCOL = 128

NC = 2
NS = 16
NW = NC * NS
EPW = E // NW
C = 80
NCHUNK = EPW // C

DRAIN_ROWS = 624

_MESH = plsc.VectorSubcoreMesh(
    core_axis_name="c", subcore_axis_name="s", num_cores=NC, num_subcores=NS
)


NPAD = 10240


def _deg_body(dst_hbm, ew_hbm, zeros_hbm, out0_hbm, out1_hbm,
              acc_sh, dst_v0, ew_v0, ssem0, dst_v1, ew_v1, ssem1):
    c = lax.axis_index("c")
    s = lax.axis_index("s")
    wid = c * NS + s

    @pl.when(s == 0)
    def _():
        pltpu.sync_copy(zeros_hbm, acc_sh)

    plsc.subcore_barrier()

    base = wid * EPW
    bufs = ((dst_v0, ew_v0, ssem0), (dst_v1, ew_v1, ssem1))

    def _wait_scatter(buf):
        dst_v, ew_v, ssem = buf
        pltpu.make_async_copy(ew_v, acc_sh.at[dst_v], ssem).wait()

    def _load(k, buf):
        dst_v, ew_v, _ = buf
        off = base + k * C
        pltpu.sync_copy(dst_hbm.at[pl.ds(off, C)], dst_v)
        pltpu.sync_copy(ew_hbm.at[pl.ds(off, C)], ew_v)

    def _step(k, cur, nxt):
        dst_v, ew_v, ssem = cur

        @pl.when(k + 1 < NCHUNK)
        def _():
            @pl.when(k >= 1)
            def _():
                _wait_scatter(nxt)

            _load(k + 1, nxt)

        pltpu.async_copy(ew_v, acc_sh.at[dst_v], ssem, add=True)

    _load(0, bufs[0])

    @pl.loop(0, NCHUNK)
    def _(k):
        @pl.when(k % 2 == 0)
        def _():
            _step(k, bufs[0], bufs[1])

        @pl.when(k % 2 == 1)
        def _():
            _step(k, bufs[1], bufs[0])

    _wait_scatter(bufs[1])
    _wait_scatter(bufs[0])

    plsc.subcore_barrier()

    @pl.when((s == 0) & (c == 0))
    def _():
        pltpu.sync_copy(acc_sh, out0_hbm)

    @pl.when((s == 0) & (c == 1))
    def _():
        pltpu.sync_copy(acc_sh, out1_hbm)


def _deg_partials(dst, ew, zeros_pad):
    return pl.kernel(
        _deg_body,
        out_type=(jax.ShapeDtypeStruct((NPAD,), jnp.float32),
                  jax.ShapeDtypeStruct((NPAD,), jnp.float32)),
        mesh=_MESH,
        scratch_types=[
            pltpu.VMEM_SHARED((NPAD,), jnp.float32),
            pltpu.VMEM((C,), jnp.int32),
            pltpu.VMEM((C,), jnp.float32),
            pltpu.SemaphoreType.DMA,
            pltpu.VMEM((C,), jnp.int32),
            pltpu.VMEM((C,), jnp.float32),
            pltpu.SemaphoreType.DMA,
        ],
    )(dst, ew, zeros_pad)


def _spmm_body(g_hbm, src_hbm, dst_hbm, ew_hbm, zeros_hbm, out_hbm,
               acc_sh, src_big, ew_big,
               dst_v0, rows_v0, gsem0, ssem0,
               dst_v1, rows_v1, gsem1, ssem1):
    c = lax.axis_index("c")
    s = lax.axis_index("s")
    wid = c * NS + s

    @pl.when(s == 0)
    def _():
        pltpu.sync_copy(zeros_hbm, acc_sh)

    base = wid * EPW
    pltpu.sync_copy(src_hbm.at[pl.ds(base, EPW)], src_big)
    pltpu.sync_copy(ew_hbm.at[pl.ds(base, EPW)], ew_big)

    plsc.subcore_barrier()

    bufs = ((dst_v0, rows_v0, gsem0, ssem0),
            (dst_v1, rows_v1, gsem1, ssem1))

    def _wait_scatter(buf):
        dst_v, rows_v, _, ssem = buf
        pltpu.make_async_copy(rows_v, acc_sh.at[dst_v], ssem).wait()

    def _start_gather(k, buf):
        dst_v, rows_v, gsem, _ = buf
        pltpu.sync_copy(dst_hbm.at[pl.ds(base + k * C, C)], dst_v)
        pltpu.async_copy(g_hbm.at[src_big.at[pl.ds(k * C, C)]], rows_v, gsem)

    def _step(k, cur, nxt):
        dst_v, rows_v, gsem, ssem = cur
        pltpu.make_async_copy(g_hbm.at[src_big.at[pl.ds(k * C, C)]],
                              rows_v, gsem).wait()

        @pl.when(k + 1 < NCHUNK)
        def _():
            @pl.when(k >= 1)
            def _():
                _wait_scatter(nxt)

            _start_gather(k + 1, nxt)

        @pl.loop(0, C // 16)
        def _(gidx):
            wvec = ew_big[pl.ds(k * C + gidx * 16, 16)]
            for lane in range(16):
                w = jnp.full((16,), wvec[lane])
                r = gidx * 16 + lane
                for j in range(DCOL // 16):
                    sl = pl.ds(j * 16, 16)
                    rows_v[r, sl] = rows_v[r, sl] * w

        pltpu.async_copy(rows_v, acc_sh.at[dst_v], ssem, add=True)

    _start_gather(0, bufs[0])

    @pl.loop(0, NCHUNK)
    def _(k):
        for par in range(2):
            @pl.when(k % 2 == par)
            def _(par=par):
                _step(k, bufs[par], bufs[1 - par])

    _wait_scatter(bufs[(NCHUNK - 2) % 2])
    _wait_scatter(bufs[(NCHUNK - 1) % 2])

    plsc.subcore_barrier()

    row0 = s * DRAIN_ROWS

    @pl.when(s < NS - 1)
    def _():
        pltpu.sync_copy(acc_sh.at[pl.ds(row0, DRAIN_ROWS)],
                        out_hbm.at[c, pl.ds(row0, DRAIN_ROWS)])

    @pl.when(s == NS - 1)
    def _():
        last = (NS - 1) * DRAIN_ROWS
        pltpu.sync_copy(acc_sh.at[pl.ds(last, N - last)],
                        out_hbm.at[c, pl.ds(last, N - last)])


def _spmm_partials(g, src, dst, ew, zeros_nd):
    ring = []
    for _ in range(2):
        ring += [
            pltpu.VMEM((C,), jnp.int32),
            pltpu.VMEM((C, DCOL), jnp.float32),
            pltpu.SemaphoreType.DMA,
            pltpu.SemaphoreType.DMA,
        ]
    return pl.kernel(
        _spmm_body,
        out_type=jax.ShapeDtypeStruct((NC, N, DCOL), jnp.float32),
        mesh=_MESH,
        scratch_types=[
            pltpu.VMEM_SHARED((N, DCOL), jnp.float32),
            pltpu.VMEM((EPW,), jnp.int32),
            pltpu.VMEM((EPW,), jnp.float32),
        ] + ring,
    )(g, src, dst, ew, zeros_nd)


BLK = 1000


def _mm1_body(degp_ref, x_ref, w_ref, dis_ref, ga_ref, gb_ref):
    deg = degp_ref[0] + degp_ref[1] + 1.0
    dis = lax.rsqrt(deg)
    h = jnp.dot(x_ref[...], w_ref[...],
                preferred_element_type=jnp.float32,
                precision=lax.Precision.HIGHEST)
    g = h * dis
    dis_ref[...] = dis
    ga_ref[...] = g[:, :DCOL]
    gb_ref[...] = g[:, DCOL:]


def _mm1(degp, x, W1):
    return pl.pallas_call(
        _mm1_body,
        grid=(N // BLK,),
        in_specs=[
            pl.BlockSpec((NC, BLK, 1), lambda i: (0, i, 0)),
            pl.BlockSpec((BLK, D_IN), lambda i: (i, 0)),
            pl.BlockSpec((D_IN, D_HID), lambda i: (0, 0)),
        ],
        out_specs=[
            pl.BlockSpec((BLK, 1), lambda i: (i, 0)),
            pl.BlockSpec((BLK, DCOL), lambda i: (i, 0)),
            pl.BlockSpec((BLK, DCOL), lambda i: (i, 0)),
        ],
        out_shape=[
            jax.ShapeDtypeStruct((N, 1), jnp.float32),
            jax.ShapeDtypeStruct((N, DCOL), jnp.float32),
            jax.ShapeDtypeStruct((N, DCOL), jnp.float32),
        ],
    )(degp, x, W1)


def _mid_body(s1a_ref, s1b_ref, ga_ref, gb_ref, dis_ref, b1_ref, w2_ref,
              g2_ref):
    dis = dis_ref[...]
    b1 = b1_ref[...]
    za = dis * (s1a_ref[0] + s1a_ref[1] + ga_ref[...]) + b1[:, :DCOL]
    zb = dis * (s1b_ref[0] + s1b_ref[1] + gb_ref[...]) + b1[:, DCOL:]
    z = jnp.maximum(jnp.concatenate([za, zb], axis=1), 0.0)
    h2 = jnp.dot(z, w2_ref[...],
                 preferred_element_type=jnp.float32,
                 precision=lax.Precision.HIGHEST)
    g2_ref[...] = h2 * dis


def _mid(s1a, s1b, ga, gb, dis, b1, W2):
    return pl.pallas_call(
        _mid_body,
        grid=(N // BLK,),
        in_specs=[
            pl.BlockSpec((NC, BLK, DCOL), lambda i: (0, i, 0)),
            pl.BlockSpec((NC, BLK, DCOL), lambda i: (0, i, 0)),
            pl.BlockSpec((BLK, DCOL), lambda i: (i, 0)),
            pl.BlockSpec((BLK, DCOL), lambda i: (i, 0)),
            pl.BlockSpec((BLK, 1), lambda i: (i, 0)),
            pl.BlockSpec((1, D_HID), lambda i: (0, 0)),
            pl.BlockSpec((D_HID, D_OUT), lambda i: (0, 0)),
        ],
        out_specs=pl.BlockSpec((BLK, D_OUT), lambda i: (i, 0)),
        out_shape=jax.ShapeDtypeStruct((N, D_OUT), jnp.float32),
    )(s1a, s1b, ga, gb, dis, b1, W2)


def _fin_body(s2_ref, g2_ref, dis_ref, b2_ref, out_ref):
    dis = dis_ref[...]
    out_ref[...] = (dis * (s2_ref[0] + s2_ref[1] + g2_ref[...])
                    + b2_ref[...])


def _fin(s2, g2, dis, b2):
    return pl.pallas_call(
        _fin_body,
        grid=(N // BLK,),
        in_specs=[
            pl.BlockSpec((NC, BLK, D_OUT), lambda i: (0, i, 0)),
            pl.BlockSpec((BLK, D_OUT), lambda i: (i, 0)),
            pl.BlockSpec((BLK, 1), lambda i: (i, 0)),
            pl.BlockSpec((1, D_OUT), lambda i: (0, 0)),
        ],
        out_specs=pl.BlockSpec((BLK, D_OUT), lambda i: (i, 0)),
        out_shape=jax.ShapeDtypeStruct((N, D_OUT), jnp.float32),
    )(s2, g2, dis, b2)


def kernel(x, edge_index, edge_weight, W1, b1, W2, b2):
    ei = edge_index.astype(jnp.int32)
    src = ei[0]
    dst = ei[1]
    ew = edge_weight.astype(jnp.float32)
    zeros_nd = jnp.zeros((N, DCOL), jnp.float32)

    degp0, degp1 = _deg_partials(dst, ew, jnp.zeros((NPAD,), jnp.float32))
    degp = jnp.stack([degp0[:N], degp1[:N]]).reshape(NC, N, 1)
    dis, g1a, g1b = _mm1(degp, x, W1)
    s1a = _spmm_partials(g1a, src, dst, ew, zeros_nd)
    s1b = _spmm_partials(g1b, src, dst, ew, zeros_nd)
    g2 = _mid(s1a, s1b, g1a, g1b, dis, b1.reshape(1, D_HID), W2)
    s2 = _spmm_partials(g2, src, dst, ew, zeros_nd)
    return _fin(s2, g2, dis, b2.reshape(1, D_OUT))


# speedup vs baseline: 1.9004x; 1.3928x over previous
import functools

import jax
import jax.numpy as jnp
from jax import lax
from jax.experimental import pallas as pl
from jax.experimental.pallas import tpu as pltpu
from jax.experimental.pallas import tpu_sc as plsc

N = 10000
E = 320000
D_IN = 128
D_HID = 256
D_OUT = 128
DCOL = 128

NC = 2
NS = 16
NW = NC * NS
EPW = E // NW
C = 80
NCHUNK = EPW // C

DRAIN_ROWS = 624

_MESH = plsc.VectorSubcoreMesh(
    core_axis_name="c", subcore_axis_name="s", num_cores=NC, num_subcores=NS
)


NPAD = 10240


def _deg_body(dst_hbm, ew_hbm, zeros_hbm, out0_hbm, out1_hbm,
              acc_sh, dst_v0, ew_v0, lsem0, ssem0,
              dst_v1, ew_v1, lsem1, ssem1):
    c = lax.axis_index("c")
    s = lax.axis_index("s")
    wid = c * NS + s

    zrows = NPAD // NS
    pltpu.sync_copy(zeros_hbm.at[pl.ds(s * zrows, zrows)],
                    acc_sh.at[pl.ds(s * zrows, zrows)])

    plsc.subcore_barrier()

    base = wid * EPW
    bufs = ((dst_v0, ew_v0, lsem0, ssem0), (dst_v1, ew_v1, lsem1, ssem1))

    def _wait_scatter(buf):
        dst_v, ew_v, _, ssem = buf
        pltpu.make_async_copy(ew_v, acc_sh.at[dst_v], ssem).wait()

    def _load(k, buf):
        dst_v, ew_v, lsem, _ = buf
        off = base + k * C
        pltpu.async_copy(dst_hbm.at[pl.ds(off, C)], dst_v, lsem)
        pltpu.async_copy(ew_hbm.at[pl.ds(off, C)], ew_v, lsem)

    def _wait_load(k, buf):
        dst_v, ew_v, lsem, _ = buf
        off = base + k * C
        pltpu.make_async_copy(dst_hbm.at[pl.ds(off, C)], dst_v, lsem).wait()
        pltpu.make_async_copy(ew_hbm.at[pl.ds(off, C)], ew_v, lsem).wait()

    def _step(k, cur, nxt):
        dst_v, ew_v, _, ssem = cur

        @pl.when(k + 1 < NCHUNK)
        def _():
            @pl.when(k >= 1)
            def _():
                _wait_scatter(nxt)

            _load(k + 1, nxt)

        _wait_load(k, cur)
        pltpu.async_copy(ew_v, acc_sh.at[dst_v], ssem, add=True)

    _load(0, bufs[0])

    @pl.loop(0, NCHUNK)
    def _(k):
        @pl.when(k % 2 == 0)
        def _():
            _step(k, bufs[0], bufs[1])

        @pl.when(k % 2 == 1)
        def _():
            _step(k, bufs[1], bufs[0])

    _wait_scatter(bufs[1])
    _wait_scatter(bufs[0])

    plsc.subcore_barrier()

    @pl.when((s == 0) & (c == 0))
    def _():
        pltpu.sync_copy(acc_sh, out0_hbm)

    @pl.when((s == 0) & (c == 1))
    def _():
        pltpu.sync_copy(acc_sh, out1_hbm)


def _deg_partials(dst, ew, zeros_pad):
    return pl.kernel(
        _deg_body,
        out_type=(jax.ShapeDtypeStruct((NPAD,), jnp.float32),
                  jax.ShapeDtypeStruct((NPAD,), jnp.float32)),
        mesh=_MESH,
        scratch_types=[
            pltpu.VMEM_SHARED((NPAD,), jnp.float32),
            pltpu.VMEM((C,), jnp.int32),
            pltpu.VMEM((C,), jnp.float32),
            pltpu.SemaphoreType.DMA,
            pltpu.SemaphoreType.DMA,
            pltpu.VMEM((C,), jnp.int32),
            pltpu.VMEM((C,), jnp.float32),
            pltpu.SemaphoreType.DMA,
            pltpu.SemaphoreType.DMA,
        ],
    )(dst, ew, zeros_pad)


def _spmm_body(g_hbm, src_hbm, dst_hbm, ew_hbm, zeros_hbm, out_hbm,
               acc_sh, src_big, ew_big,
               dst_v0, rows_v0, lsem0, gsem0, ssem0,
               dst_v1, rows_v1, lsem1, gsem1, ssem1):
    c = lax.axis_index("c")
    s = lax.axis_index("s")
    wid = c * NS + s

    @pl.when(s < NS - 1)
    def _():
        pltpu.sync_copy(zeros_hbm.at[pl.ds(s * DRAIN_ROWS, DRAIN_ROWS)],
                        acc_sh.at[pl.ds(s * DRAIN_ROWS, DRAIN_ROWS)])

    @pl.when(s == NS - 1)
    def _():
        last = (NS - 1) * DRAIN_ROWS
        pltpu.sync_copy(zeros_hbm.at[pl.ds(last, N - last)],
                        acc_sh.at[pl.ds(last, N - last)])

    base = wid * EPW
    pltpu.sync_copy(src_hbm.at[pl.ds(base, EPW)], src_big)
    pltpu.sync_copy(ew_hbm.at[pl.ds(base, EPW)], ew_big)

    plsc.subcore_barrier()

    bufs = ((dst_v0, rows_v0, lsem0, gsem0, ssem0),
            (dst_v1, rows_v1, lsem1, gsem1, ssem1))

    def _wait_scatter(buf):
        dst_v, rows_v, _, _, ssem = buf
        pltpu.make_async_copy(rows_v, acc_sh.at[dst_v], ssem).wait()

    def _start_gather(k, buf):
        dst_v, rows_v, lsem, gsem, _ = buf
        pltpu.async_copy(dst_hbm.at[pl.ds(base + k * C, C)], dst_v, lsem)
        pltpu.async_copy(g_hbm.at[src_big.at[pl.ds(k * C, C)]], rows_v, gsem)

    def _step(k, cur, nxt):
        dst_v, rows_v, lsem, gsem, ssem = cur
        pltpu.make_async_copy(g_hbm.at[src_big.at[pl.ds(k * C, C)]],
                              rows_v, gsem).wait()

        @pl.when(k + 1 < NCHUNK)
        def _():
            @pl.when(k >= 1)
            def _():
                _wait_scatter(nxt)

            _start_gather(k + 1, nxt)

        @pl.loop(0, C // 16)
        def _(gidx):
            wvec = ew_big[pl.ds(k * C + gidx * 16, 16)]
            for lane in range(16):
                w = jnp.full((16,), wvec[lane])
                r = gidx * 16 + lane
                for j in range(DCOL // 16):
                    sl = pl.ds(j * 16, 16)
                    rows_v[r, sl] = rows_v[r, sl] * w

        pltpu.make_async_copy(dst_hbm.at[pl.ds(base + k * C, C)],
                              dst_v, lsem).wait()
        pltpu.async_copy(rows_v, acc_sh.at[dst_v], ssem, add=True)

    _start_gather(0, bufs[0])

    @pl.loop(0, NCHUNK)
    def _(k):
        for par in range(2):
            @pl.when(k % 2 == par)
            def _(par=par):
                _step(k, bufs[par], bufs[1 - par])

    _wait_scatter(bufs[(NCHUNK - 2) % 2])
    _wait_scatter(bufs[(NCHUNK - 1) % 2])

    plsc.subcore_barrier()

    row0 = s * DRAIN_ROWS

    @pl.when(s < NS - 1)
    def _():
        pltpu.sync_copy(acc_sh.at[pl.ds(row0, DRAIN_ROWS)],
                        out_hbm.at[c, pl.ds(row0, DRAIN_ROWS)])

    @pl.when(s == NS - 1)
    def _():
        last = (NS - 1) * DRAIN_ROWS
        pltpu.sync_copy(acc_sh.at[pl.ds(last, N - last)],
                        out_hbm.at[c, pl.ds(last, N - last)])


def _spmm_partials(g, src, dst, ew, zeros_nd):
    ring = []
    for _ in range(2):
        ring += [
            pltpu.VMEM((C,), jnp.int32),
            pltpu.VMEM((C, DCOL), jnp.float32),
            pltpu.SemaphoreType.DMA,
            pltpu.SemaphoreType.DMA,
            pltpu.SemaphoreType.DMA,
        ]
    return pl.kernel(
        _spmm_body,
        out_type=jax.ShapeDtypeStruct((NC, N, DCOL), jnp.float32),
        mesh=_MESH,
        scratch_types=[
            pltpu.VMEM_SHARED((N, DCOL), jnp.float32),
            pltpu.VMEM((EPW,), jnp.int32),
            pltpu.VMEM((EPW,), jnp.float32),
        ] + ring,
    )(g, src, dst, ew, zeros_nd)


BLK = 1000


def _mm1_body(degp_ref, x_ref, w_ref, dis_ref, ga_ref, gb_ref):
    deg = degp_ref[0] + degp_ref[1] + 1.0
    dis = lax.rsqrt(deg)
    h = jnp.dot(x_ref[...], w_ref[...],
                preferred_element_type=jnp.float32,
                precision=lax.Precision.HIGHEST)
    g = h * dis
    dis_ref[...] = dis
    ga_ref[...] = g[:, :DCOL]
    gb_ref[...] = g[:, DCOL:]


def _mm1(degp, x, W1):
    return pl.pallas_call(
        _mm1_body,
        grid=(N // BLK,),
        in_specs=[
            pl.BlockSpec((NC, BLK, 1), lambda i: (0, i, 0)),
            pl.BlockSpec((BLK, D_IN), lambda i: (i, 0)),
            pl.BlockSpec((D_IN, D_HID), lambda i: (0, 0)),
        ],
        out_specs=[
            pl.BlockSpec((BLK, 1), lambda i: (i, 0)),
            pl.BlockSpec((BLK, DCOL), lambda i: (i, 0)),
            pl.BlockSpec((BLK, DCOL), lambda i: (i, 0)),
        ],
        out_shape=[
            jax.ShapeDtypeStruct((N, 1), jnp.float32),
            jax.ShapeDtypeStruct((N, DCOL), jnp.float32),
            jax.ShapeDtypeStruct((N, DCOL), jnp.float32),
        ],
    )(degp, x, W1)


def _mid_body(s1a_ref, s1b_ref, ga_ref, gb_ref, dis_ref, b1_ref, w2_ref,
              g2_ref):
    dis = dis_ref[...]
    b1 = b1_ref[...]
    za = dis * (s1a_ref[0] + s1a_ref[1] + ga_ref[...]) + b1[:, :DCOL]
    zb = dis * (s1b_ref[0] + s1b_ref[1] + gb_ref[...]) + b1[:, DCOL:]
    z = jnp.maximum(jnp.concatenate([za, zb], axis=1), 0.0)
    h2 = jnp.dot(z, w2_ref[...],
                 preferred_element_type=jnp.float32,
                 precision=lax.Precision.HIGHEST)
    g2_ref[...] = h2 * dis


def _mid(s1a, s1b, ga, gb, dis, b1, W2):
    return pl.pallas_call(
        _mid_body,
        grid=(N // BLK,),
        in_specs=[
            pl.BlockSpec((NC, BLK, DCOL), lambda i: (0, i, 0)),
            pl.BlockSpec((NC, BLK, DCOL), lambda i: (0, i, 0)),
            pl.BlockSpec((BLK, DCOL), lambda i: (i, 0)),
            pl.BlockSpec((BLK, DCOL), lambda i: (i, 0)),
            pl.BlockSpec((BLK, 1), lambda i: (i, 0)),
            pl.BlockSpec((1, D_HID), lambda i: (0, 0)),
            pl.BlockSpec((D_HID, D_OUT), lambda i: (0, 0)),
        ],
        out_specs=pl.BlockSpec((BLK, D_OUT), lambda i: (i, 0)),
        out_shape=jax.ShapeDtypeStruct((N, D_OUT), jnp.float32),
    )(s1a, s1b, ga, gb, dis, b1, W2)


def _fin_body(s2_ref, g2_ref, dis_ref, b2_ref, out_ref):
    dis = dis_ref[...]
    out_ref[...] = (dis * (s2_ref[0] + s2_ref[1] + g2_ref[...])
                    + b2_ref[...])


def _fin(s2, g2, dis, b2):
    return pl.pallas_call(
        _fin_body,
        grid=(N // BLK,),
        in_specs=[
            pl.BlockSpec((NC, BLK, D_OUT), lambda i: (0, i, 0)),
            pl.BlockSpec((BLK, D_OUT), lambda i: (i, 0)),
            pl.BlockSpec((BLK, 1), lambda i: (i, 0)),
            pl.BlockSpec((1, D_OUT), lambda i: (0, 0)),
        ],
        out_specs=pl.BlockSpec((BLK, D_OUT), lambda i: (i, 0)),
        out_shape=jax.ShapeDtypeStruct((N, D_OUT), jnp.float32),
    )(s2, g2, dis, b2)


def kernel(x, edge_index, edge_weight, W1, b1, W2, b2):
    ei = edge_index.astype(jnp.int32)
    src = ei[0]
    dst = ei[1]
    ew = edge_weight.astype(jnp.float32)
    zeros_nd = jnp.zeros((N, DCOL), jnp.float32)

    degp0, degp1 = _deg_partials(dst, ew, jnp.zeros((NPAD,), jnp.float32))
    degp = jnp.stack([degp0[:N], degp1[:N]]).reshape(NC, N, 1)
    dis, g1a, g1b = _mm1(degp, x, W1)
    s1a = _spmm_partials(g1a, src, dst, ew, zeros_nd)
    s1b = _spmm_partials(g1b, src, dst, ew, zeros_nd)
    g2 = _mid(s1a, s1b, g1a, g1b, dis, b1.reshape(1, D_HID), W2)
    s2 = _spmm_partials(g2, src, dst, ew, zeros_nd)
    return _fin(s2, g2, dis, b2.reshape(1, D_OUT))


# speedup vs baseline: 2.3098x; 1.2154x over previous
import functools

import jax
import jax.numpy as jnp
from jax import lax
from jax.experimental import pallas as pl
from jax.experimental.pallas import tpu as pltpu
from jax.experimental.pallas import tpu_sc as plsc

N = 10000
E = 320000
D_IN = 128
D_HID = 256
D_OUT = 128
DCOL = 128

NC = 2
NS = 16
NW = NC * NS
EPW = E // NW
C = 80
NCHUNK = EPW // C

DRAIN_ROWS = 624

_MESH = plsc.VectorSubcoreMesh(
    core_axis_name="c", subcore_axis_name="s", num_cores=NC, num_subcores=NS
)


NPAD = 10240


def _deg_body(dst_hbm, ew_hbm, zeros_hbm, out0_hbm, out1_hbm,
              acc_sh, dst_v0, ew_v0, lsem0, ssem0,
              dst_v1, ew_v1, lsem1, ssem1):
    c = lax.axis_index("c")
    s = lax.axis_index("s")
    wid = c * NS + s

    zrows = NPAD // NS
    pltpu.sync_copy(zeros_hbm.at[pl.ds(s * zrows, zrows)],
                    acc_sh.at[pl.ds(s * zrows, zrows)])

    plsc.subcore_barrier()

    base = wid * EPW
    bufs = ((dst_v0, ew_v0, lsem0, ssem0), (dst_v1, ew_v1, lsem1, ssem1))

    def _wait_scatter(buf):
        dst_v, ew_v, _, ssem = buf
        pltpu.make_async_copy(ew_v, acc_sh.at[dst_v], ssem).wait()

    def _load(k, buf):
        dst_v, ew_v, lsem, _ = buf
        off = base + k * C
        pltpu.async_copy(dst_hbm.at[pl.ds(off, C)], dst_v, lsem)
        pltpu.async_copy(ew_hbm.at[pl.ds(off, C)], ew_v, lsem)

    def _wait_load(k, buf):
        dst_v, ew_v, lsem, _ = buf
        off = base + k * C
        pltpu.make_async_copy(dst_hbm.at[pl.ds(off, C)], dst_v, lsem).wait()
        pltpu.make_async_copy(ew_hbm.at[pl.ds(off, C)], ew_v, lsem).wait()

    def _step(k, cur, nxt):
        dst_v, ew_v, _, ssem = cur

        @pl.when(k + 1 < NCHUNK)
        def _():
            @pl.when(k >= 1)
            def _():
                _wait_scatter(nxt)

            _load(k + 1, nxt)

        _wait_load(k, cur)
        pltpu.async_copy(ew_v, acc_sh.at[dst_v], ssem, add=True)

    _load(0, bufs[0])

    @pl.loop(0, NCHUNK)
    def _(k):
        @pl.when(k % 2 == 0)
        def _():
            _step(k, bufs[0], bufs[1])

        @pl.when(k % 2 == 1)
        def _():
            _step(k, bufs[1], bufs[0])

    _wait_scatter(bufs[1])
    _wait_scatter(bufs[0])

    plsc.subcore_barrier()

    @pl.when((s == 0) & (c == 0))
    def _():
        pltpu.sync_copy(acc_sh, out0_hbm)

    @pl.when((s == 0) & (c == 1))
    def _():
        pltpu.sync_copy(acc_sh, out1_hbm)


def _deg_partials(dst, ew, zeros_pad):
    return pl.kernel(
        _deg_body,
        out_type=(jax.ShapeDtypeStruct((NPAD,), jnp.float32),
                  jax.ShapeDtypeStruct((NPAD,), jnp.float32)),
        mesh=_MESH,
        scratch_types=[
            pltpu.VMEM_SHARED((NPAD,), jnp.float32),
            pltpu.VMEM((C,), jnp.int32),
            pltpu.VMEM((C,), jnp.float32),
            pltpu.SemaphoreType.DMA,
            pltpu.SemaphoreType.DMA,
            pltpu.VMEM((C,), jnp.int32),
            pltpu.VMEM((C,), jnp.float32),
            pltpu.SemaphoreType.DMA,
            pltpu.SemaphoreType.DMA,
        ],
    )(dst, ew, zeros_pad)


def _spmm_body(g_hbm, src_hbm, dst_hbm, ew_hbm, zeros_hbm, out_hbm,
               acc_sh, src_big,
               dst_v0, ew_v0, rows_v0, lsem0, gsem0, ssem0,
               dst_v1, ew_v1, rows_v1, lsem1, gsem1, ssem1,
               dst_v2, ew_v2, rows_v2, lsem2, gsem2, ssem2):
    c = lax.axis_index("c")
    s = lax.axis_index("s")
    wid = c * NS + s

    @pl.when(s < NS - 1)
    def _():
        pltpu.sync_copy(zeros_hbm.at[pl.ds(s * DRAIN_ROWS, DRAIN_ROWS)],
                        acc_sh.at[pl.ds(s * DRAIN_ROWS, DRAIN_ROWS)])

    @pl.when(s == NS - 1)
    def _():
        last = (NS - 1) * DRAIN_ROWS
        pltpu.sync_copy(zeros_hbm.at[pl.ds(last, N - last)],
                        acc_sh.at[pl.ds(last, N - last)])

    base = wid * EPW
    pltpu.sync_copy(src_hbm.at[pl.ds(base, EPW)], src_big)

    plsc.subcore_barrier()

    bufs = ((dst_v0, ew_v0, rows_v0, lsem0, gsem0, ssem0),
            (dst_v1, ew_v1, rows_v1, lsem1, gsem1, ssem1),
            (dst_v2, ew_v2, rows_v2, lsem2, gsem2, ssem2))

    def _wait_scatter(buf):
        dst_v, _, rows_v, _, _, ssem = buf
        pltpu.make_async_copy(rows_v, acc_sh.at[dst_v], ssem).wait()

    def _start_gather(k, buf):
        dst_v, ew_v, rows_v, lsem, gsem, _ = buf
        pltpu.async_copy(dst_hbm.at[pl.ds(base + k * C, C)], dst_v, lsem)
        pltpu.async_copy(ew_hbm.at[pl.ds(base + k * C, C)], ew_v, lsem)
        pltpu.async_copy(g_hbm.at[src_big.at[pl.ds(k * C, C)]], rows_v, gsem)

    def _step(k, cur, nxt):
        dst_v, ew_v, rows_v, lsem, gsem, ssem = cur
        pltpu.make_async_copy(g_hbm.at[src_big.at[pl.ds(k * C, C)]],
                              rows_v, gsem).wait()

        @pl.when(k + 2 < NCHUNK)
        def _():
            @pl.when(k >= 1)
            def _():
                _wait_scatter(nxt)

            _start_gather(k + 2, nxt)

        pltpu.make_async_copy(ew_hbm.at[pl.ds(base + k * C, C)],
                              ew_v, lsem).wait()

        @pl.loop(0, C // 16)
        def _(gidx):
            wvec = ew_v[pl.ds(gidx * 16, 16)]
            for lane in range(16):
                w = jnp.full((16,), wvec[lane])
                r = gidx * 16 + lane
                for j in range(DCOL // 16):
                    sl = pl.ds(j * 16, 16)
                    rows_v[r, sl] = rows_v[r, sl] * w

        pltpu.make_async_copy(dst_hbm.at[pl.ds(base + k * C, C)],
                              dst_v, lsem).wait()
        pltpu.async_copy(rows_v, acc_sh.at[dst_v], ssem, add=True)

    _start_gather(0, bufs[0])
    _start_gather(1, bufs[1])

    @pl.loop(0, NCHUNK)
    def _(k):
        for par in range(3):
            @pl.when(k % 3 == par)
            def _(par=par):
                _step(k, bufs[par], bufs[(par + 2) % 3])

    _wait_scatter(bufs[(NCHUNK - 3) % 3])
    _wait_scatter(bufs[(NCHUNK - 2) % 3])
    _wait_scatter(bufs[(NCHUNK - 1) % 3])

    plsc.subcore_barrier()

    row0 = s * DRAIN_ROWS

    @pl.when(s < NS - 1)
    def _():
        pltpu.sync_copy(acc_sh.at[pl.ds(row0, DRAIN_ROWS)],
                        out_hbm.at[c, pl.ds(row0, DRAIN_ROWS)])

    @pl.when(s == NS - 1)
    def _():
        last = (NS - 1) * DRAIN_ROWS
        pltpu.sync_copy(acc_sh.at[pl.ds(last, N - last)],
                        out_hbm.at[c, pl.ds(last, N - last)])


def _spmm_partials(g, src, dst, ew, zeros_nd):
    ring = []
    for _ in range(3):
        ring += [
            pltpu.VMEM((C,), jnp.int32),
            pltpu.VMEM((C,), jnp.float32),
            pltpu.VMEM((C, DCOL), jnp.float32),
            pltpu.SemaphoreType.DMA,
            pltpu.SemaphoreType.DMA,
            pltpu.SemaphoreType.DMA,
        ]
    return pl.kernel(
        _spmm_body,
        out_type=jax.ShapeDtypeStruct((NC, N, DCOL), jnp.float32),
        mesh=_MESH,
        scratch_types=[
            pltpu.VMEM_SHARED((N, DCOL), jnp.float32),
            pltpu.VMEM((EPW,), jnp.int32),
        ] + ring,
    )(g, src, dst, ew, zeros_nd)


BLK = 1000


def _mm1_body(degp_ref, x_ref, w_ref, dis_ref, ga_ref, gb_ref):
    deg = degp_ref[0] + degp_ref[1] + 1.0
    dis = lax.rsqrt(deg)
    h = jnp.dot(x_ref[...], w_ref[...],
                preferred_element_type=jnp.float32,
                precision=lax.Precision.HIGHEST)
    g = h * dis
    dis_ref[...] = dis
    ga_ref[...] = g[:, :DCOL]
    gb_ref[...] = g[:, DCOL:]


def _mm1(degp, x, W1):
    return pl.pallas_call(
        _mm1_body,
        grid=(N // BLK,),
        in_specs=[
            pl.BlockSpec((NC, BLK, 1), lambda i: (0, i, 0)),
            pl.BlockSpec((BLK, D_IN), lambda i: (i, 0)),
            pl.BlockSpec((D_IN, D_HID), lambda i: (0, 0)),
        ],
        out_specs=[
            pl.BlockSpec((BLK, 1), lambda i: (i, 0)),
            pl.BlockSpec((BLK, DCOL), lambda i: (i, 0)),
            pl.BlockSpec((BLK, DCOL), lambda i: (i, 0)),
        ],
        out_shape=[
            jax.ShapeDtypeStruct((N, 1), jnp.float32),
            jax.ShapeDtypeStruct((N, DCOL), jnp.float32),
            jax.ShapeDtypeStruct((N, DCOL), jnp.float32),
        ],
    )(degp, x, W1)


def _mid_body(s1a_ref, s1b_ref, ga_ref, gb_ref, dis_ref, b1_ref, w2_ref,
              g2_ref):
    dis = dis_ref[...]
    b1 = b1_ref[...]
    za = dis * (s1a_ref[0] + s1a_ref[1] + ga_ref[...]) + b1[:, :DCOL]
    zb = dis * (s1b_ref[0] + s1b_ref[1] + gb_ref[...]) + b1[:, DCOL:]
    z = jnp.maximum(jnp.concatenate([za, zb], axis=1), 0.0)
    h2 = jnp.dot(z, w2_ref[...],
                 preferred_element_type=jnp.float32,
                 precision=lax.Precision.HIGHEST)
    g2_ref[...] = h2 * dis


def _mid(s1a, s1b, ga, gb, dis, b1, W2):
    return pl.pallas_call(
        _mid_body,
        grid=(N // BLK,),
        in_specs=[
            pl.BlockSpec((NC, BLK, DCOL), lambda i: (0, i, 0)),
            pl.BlockSpec((NC, BLK, DCOL), lambda i: (0, i, 0)),
            pl.BlockSpec((BLK, DCOL), lambda i: (i, 0)),
            pl.BlockSpec((BLK, DCOL), lambda i: (i, 0)),
            pl.BlockSpec((BLK, 1), lambda i: (i, 0)),
            pl.BlockSpec((1, D_HID), lambda i: (0, 0)),
            pl.BlockSpec((D_HID, D_OUT), lambda i: (0, 0)),
        ],
        out_specs=pl.BlockSpec((BLK, D_OUT), lambda i: (i, 0)),
        out_shape=jax.ShapeDtypeStruct((N, D_OUT), jnp.float32),
    )(s1a, s1b, ga, gb, dis, b1, W2)


def _fin_body(s2_ref, g2_ref, dis_ref, b2_ref, out_ref):
    dis = dis_ref[...]
    out_ref[...] = (dis * (s2_ref[0] + s2_ref[1] + g2_ref[...])
                    + b2_ref[...])


def _fin(s2, g2, dis, b2):
    return pl.pallas_call(
        _fin_body,
        grid=(N // BLK,),
        in_specs=[
            pl.BlockSpec((NC, BLK, D_OUT), lambda i: (0, i, 0)),
            pl.BlockSpec((BLK, D_OUT), lambda i: (i, 0)),
            pl.BlockSpec((BLK, 1), lambda i: (i, 0)),
            pl.BlockSpec((1, D_OUT), lambda i: (0, 0)),
        ],
        out_specs=pl.BlockSpec((BLK, D_OUT), lambda i: (i, 0)),
        out_shape=jax.ShapeDtypeStruct((N, D_OUT), jnp.float32),
    )(s2, g2, dis, b2)


def kernel(x, edge_index, edge_weight, W1, b1, W2, b2):
    ei = edge_index.astype(jnp.int32)
    src = ei[0]
    dst = ei[1]
    ew = edge_weight.astype(jnp.float32)
    zeros_nd = jnp.zeros((N, DCOL), jnp.float32)

    degp0, degp1 = _deg_partials(dst, ew, jnp.zeros((NPAD,), jnp.float32))
    degp = jnp.stack([degp0[:N], degp1[:N]]).reshape(NC, N, 1)
    dis, g1a, g1b = _mm1(degp, x, W1)
    s1a = _spmm_partials(g1a, src, dst, ew, zeros_nd)
    s1b = _spmm_partials(g1b, src, dst, ew, zeros_nd)
    g2 = _mid(s1a, s1b, g1a, g1b, dis, b1.reshape(1, D_HID), W2)
    s2 = _spmm_partials(g2, src, dst, ew, zeros_nd)
    return _fin(s2, g2, dis, b2.reshape(1, D_OUT))


# speedup vs baseline: 2.3545x; 1.0194x over previous
import functools

import jax
import jax.numpy as jnp
from jax import lax
from jax.experimental import pallas as pl
from jax.experimental.pallas import tpu as pltpu
from jax.experimental.pallas import tpu_sc as plsc

N = 10000
E = 320000
D_IN = 128
D_HID = 256
D_OUT = 128
DCOL = 128

NC = 2
NS = 16
NW = NC * NS
EPW = E // NW
C = 80
NCHUNK = EPW // C

DRAIN_ROWS = 624

_MESH = plsc.VectorSubcoreMesh(
    core_axis_name="c", subcore_axis_name="s", num_cores=NC, num_subcores=NS
)


NPAD = 10240


def _deg_body(dst_hbm, ew_hbm, zeros_hbm, out0_hbm, out1_hbm,
              acc_sh, dst_v0, ew_v0, lsem0, ssem0,
              dst_v1, ew_v1, lsem1, ssem1):
    c = lax.axis_index("c")
    s = lax.axis_index("s")
    wid = c * NS + s

    zrows = NPAD // NS
    pltpu.sync_copy(zeros_hbm.at[pl.ds(s * zrows, zrows)],
                    acc_sh.at[pl.ds(s * zrows, zrows)])

    plsc.subcore_barrier()

    base = wid * EPW
    bufs = ((dst_v0, ew_v0, lsem0, ssem0), (dst_v1, ew_v1, lsem1, ssem1))

    def _wait_scatter(buf):
        dst_v, ew_v, _, ssem = buf
        pltpu.make_async_copy(ew_v, acc_sh.at[dst_v], ssem).wait()

    def _load(k, buf):
        dst_v, ew_v, lsem, _ = buf
        off = base + k * C
        pltpu.async_copy(dst_hbm.at[pl.ds(off, C)], dst_v, lsem)
        pltpu.async_copy(ew_hbm.at[pl.ds(off, C)], ew_v, lsem)

    def _wait_load(k, buf):
        dst_v, ew_v, lsem, _ = buf
        off = base + k * C
        pltpu.make_async_copy(dst_hbm.at[pl.ds(off, C)], dst_v, lsem).wait()
        pltpu.make_async_copy(ew_hbm.at[pl.ds(off, C)], ew_v, lsem).wait()

    def _step(k, cur, nxt):
        dst_v, ew_v, _, ssem = cur

        @pl.when(k + 1 < NCHUNK)
        def _():
            @pl.when(k >= 1)
            def _():
                _wait_scatter(nxt)

            _load(k + 1, nxt)

        _wait_load(k, cur)
        pltpu.async_copy(ew_v, acc_sh.at[dst_v], ssem, add=True)

    _load(0, bufs[0])

    @pl.loop(0, NCHUNK)
    def _(k):
        @pl.when(k % 2 == 0)
        def _():
            _step(k, bufs[0], bufs[1])

        @pl.when(k % 2 == 1)
        def _():
            _step(k, bufs[1], bufs[0])

    _wait_scatter(bufs[1])
    _wait_scatter(bufs[0])

    plsc.subcore_barrier()

    @pl.when((s == 0) & (c == 0))
    def _():
        pltpu.sync_copy(acc_sh, out0_hbm)

    @pl.when((s == 0) & (c == 1))
    def _():
        pltpu.sync_copy(acc_sh, out1_hbm)


def _deg_partials(dst, ew, zeros_pad):
    return pl.kernel(
        _deg_body,
        out_type=(jax.ShapeDtypeStruct((NPAD,), jnp.float32),
                  jax.ShapeDtypeStruct((NPAD,), jnp.float32)),
        mesh=_MESH,
        scratch_types=[
            pltpu.VMEM_SHARED((NPAD,), jnp.float32),
            pltpu.VMEM((C,), jnp.int32),
            pltpu.VMEM((C,), jnp.float32),
            pltpu.SemaphoreType.DMA,
            pltpu.SemaphoreType.DMA,
            pltpu.VMEM((C,), jnp.int32),
            pltpu.VMEM((C,), jnp.float32),
            pltpu.SemaphoreType.DMA,
            pltpu.SemaphoreType.DMA,
        ],
    )(dst, ew, zeros_pad)


def _spmm_generic(g_hbms, drains, src_hbm, dst_hbm, ew_hbm, zeros_hbm,
                  acc_sh, src_big, bufs):
    c = lax.axis_index("c")
    s = lax.axis_index("s")
    wid = c * NS + s
    base = wid * EPW
    row0 = s * DRAIN_ROWS
    last = (NS - 1) * DRAIN_ROWS

    def _zero_acc():
        @pl.when(s < NS - 1)
        def _():
            pltpu.sync_copy(zeros_hbm.at[pl.ds(row0, DRAIN_ROWS)],
                            acc_sh.at[pl.ds(row0, DRAIN_ROWS)])

        @pl.when(s == NS - 1)
        def _():
            pltpu.sync_copy(zeros_hbm.at[pl.ds(last, N - last)],
                            acc_sh.at[pl.ds(last, N - last)])

    _zero_acc()
    pltpu.sync_copy(src_hbm.at[pl.ds(base, EPW)], src_big)
    plsc.subcore_barrier()

    def _wait_scatter(buf):
        dst_v, _, rows_v, _, _, ssem = buf
        pltpu.make_async_copy(rows_v, acc_sh.at[dst_v], ssem).wait()

    for h, g_hbm in enumerate(g_hbms):
        def _start_gather(k, buf, g_hbm=g_hbm):
            dst_v, ew_v, rows_v, lsem, gsem, _ = buf
            pltpu.async_copy(dst_hbm.at[pl.ds(base + k * C, C)], dst_v, lsem)
            pltpu.async_copy(ew_hbm.at[pl.ds(base + k * C, C)], ew_v, lsem)
            pltpu.async_copy(g_hbm.at[src_big.at[pl.ds(k * C, C)]],
                             rows_v, gsem)

        def _step(k, cur, nxt, g_hbm=g_hbm, _start_gather=_start_gather):
            dst_v, ew_v, rows_v, lsem, gsem, ssem = cur
            pltpu.make_async_copy(g_hbm.at[src_big.at[pl.ds(k * C, C)]],
                                  rows_v, gsem).wait()

            @pl.when(k + 2 < NCHUNK)
            def _():
                @pl.when(k >= 1)
                def _():
                    _wait_scatter(nxt)

                _start_gather(k + 2, nxt)

            pltpu.make_async_copy(ew_hbm.at[pl.ds(base + k * C, C)],
                                  ew_v, lsem).wait()

            @pl.loop(0, C // 16)
            def _(gidx):
                wvec = ew_v[pl.ds(gidx * 16, 16)]
                for lane in range(16):
                    w = jnp.full((16,), wvec[lane])
                    r = gidx * 16 + lane
                    for j in range(DCOL // 16):
                        sl = pl.ds(j * 16, 16)
                        rows_v[r, sl] = rows_v[r, sl] * w

            pltpu.make_async_copy(dst_hbm.at[pl.ds(base + k * C, C)],
                                  dst_v, lsem).wait()
            pltpu.async_copy(rows_v, acc_sh.at[dst_v], ssem, add=True)

        _start_gather(0, bufs[0])
        _start_gather(1, bufs[1])

        @pl.loop(0, NCHUNK)
        def _(k):
            for par in range(3):
                @pl.when(k % 3 == par)
                def _(par=par, _step=_step):
                    _step(k, bufs[par], bufs[(par + 2) % 3])

        _wait_scatter(bufs[(NCHUNK - 3) % 3])
        _wait_scatter(bufs[(NCHUNK - 2) % 3])
        _wait_scatter(bufs[(NCHUNK - 1) % 3])

        plsc.subcore_barrier()

        drain = drains[h]

        @pl.when(s < NS - 1)
        def _(drain=drain):
            pltpu.sync_copy(acc_sh.at[pl.ds(row0, DRAIN_ROWS)],
                            drain(pl.ds(row0, DRAIN_ROWS)))

        @pl.when(s == NS - 1)
        def _(drain=drain):
            pltpu.sync_copy(acc_sh.at[pl.ds(last, N - last)],
                            drain(pl.ds(last, N - last)))

        if h + 1 < len(g_hbms):
            _zero_acc()
            plsc.subcore_barrier()


def _spmm_body(g_hbm, src_hbm, dst_hbm, ew_hbm, zeros_hbm, out_hbm,
               acc_sh, src_big,
               dst_v0, ew_v0, rows_v0, lsem0, gsem0, ssem0,
               dst_v1, ew_v1, rows_v1, lsem1, gsem1, ssem1,
               dst_v2, ew_v2, rows_v2, lsem2, gsem2, ssem2):
    c = lax.axis_index("c")
    bufs = ((dst_v0, ew_v0, rows_v0, lsem0, gsem0, ssem0),
            (dst_v1, ew_v1, rows_v1, lsem1, gsem1, ssem1),
            (dst_v2, ew_v2, rows_v2, lsem2, gsem2, ssem2))
    _spmm_generic((g_hbm,), (lambda sl: out_hbm.at[c, sl],),
                  src_hbm, dst_hbm, ew_hbm, zeros_hbm, acc_sh, src_big, bufs)


def _spmm2_body(ga_hbm, gb_hbm, src_hbm, dst_hbm, ew_hbm, zeros_hbm, out_hbm,
                acc_sh, src_big,
                dst_v0, ew_v0, rows_v0, lsem0, gsem0, ssem0,
                dst_v1, ew_v1, rows_v1, lsem1, gsem1, ssem1,
                dst_v2, ew_v2, rows_v2, lsem2, gsem2, ssem2):
    c = lax.axis_index("c")
    bufs = ((dst_v0, ew_v0, rows_v0, lsem0, gsem0, ssem0),
            (dst_v1, ew_v1, rows_v1, lsem1, gsem1, ssem1),
            (dst_v2, ew_v2, rows_v2, lsem2, gsem2, ssem2))
    _spmm_generic((ga_hbm, gb_hbm),
                  (lambda sl: out_hbm.at[c, 0, sl],
                   lambda sl: out_hbm.at[c, 1, sl]),
                  src_hbm, dst_hbm, ew_hbm, zeros_hbm, acc_sh, src_big, bufs)


def _spmm_partials(g, src, dst, ew, zeros_nd):
    ring = []
    for _ in range(3):
        ring += [
            pltpu.VMEM((C,), jnp.int32),
            pltpu.VMEM((C,), jnp.float32),
            pltpu.VMEM((C, DCOL), jnp.float32),
            pltpu.SemaphoreType.DMA,
            pltpu.SemaphoreType.DMA,
            pltpu.SemaphoreType.DMA,
        ]
    return pl.kernel(
        _spmm_body,
        out_type=jax.ShapeDtypeStruct((NC, N, DCOL), jnp.float32),
        mesh=_MESH,
        scratch_types=[
            pltpu.VMEM_SHARED((N, DCOL), jnp.float32),
            pltpu.VMEM((EPW,), jnp.int32),
        ] + ring,
    )(g, src, dst, ew, zeros_nd)


def _spmm2_partials(ga, gb, src, dst, ew, zeros_nd):
    ring = []
    for _ in range(3):
        ring += [
            pltpu.VMEM((C,), jnp.int32),
            pltpu.VMEM((C,), jnp.float32),
            pltpu.VMEM((C, DCOL), jnp.float32),
            pltpu.SemaphoreType.DMA,
            pltpu.SemaphoreType.DMA,
            pltpu.SemaphoreType.DMA,
        ]
    return pl.kernel(
        _spmm2_body,
        out_type=jax.ShapeDtypeStruct((NC, 2, N, DCOL), jnp.float32),
        mesh=_MESH,
        scratch_types=[
            pltpu.VMEM_SHARED((N, DCOL), jnp.float32),
            pltpu.VMEM((EPW,), jnp.int32),
        ] + ring,
    )(ga, gb, src, dst, ew, zeros_nd)


BLK = 1000


def _mm1_body(degp_ref, x_ref, w_ref, dis_ref, ga_ref, gb_ref):
    deg = degp_ref[0] + degp_ref[1] + 1.0
    dis = lax.rsqrt(deg)
    h = jnp.dot(x_ref[...], w_ref[...],
                preferred_element_type=jnp.float32,
                precision=lax.Precision.HIGHEST)
    g = h * dis
    dis_ref[...] = dis
    ga_ref[...] = g[:, :DCOL]
    gb_ref[...] = g[:, DCOL:]


def _mm1(degp, x, W1):
    return pl.pallas_call(
        _mm1_body,
        grid=(N // BLK,),
        in_specs=[
            pl.BlockSpec((NC, BLK, 1), lambda i: (0, i, 0)),
            pl.BlockSpec((BLK, D_IN), lambda i: (i, 0)),
            pl.BlockSpec((D_IN, D_HID), lambda i: (0, 0)),
        ],
        out_specs=[
            pl.BlockSpec((BLK, 1), lambda i: (i, 0)),
            pl.BlockSpec((BLK, DCOL), lambda i: (i, 0)),
            pl.BlockSpec((BLK, DCOL), lambda i: (i, 0)),
        ],
        out_shape=[
            jax.ShapeDtypeStruct((N, 1), jnp.float32),
            jax.ShapeDtypeStruct((N, DCOL), jnp.float32),
            jax.ShapeDtypeStruct((N, DCOL), jnp.float32),
        ],
    )(degp, x, W1)


def _mid_body(s1_ref, ga_ref, gb_ref, dis_ref, b1_ref, w2_ref, g2_ref):
    dis = dis_ref[...]
    b1 = b1_ref[...]
    za = dis * (s1_ref[0, 0] + s1_ref[1, 0] + ga_ref[...]) + b1[:, :DCOL]
    zb = dis * (s1_ref[0, 1] + s1_ref[1, 1] + gb_ref[...]) + b1[:, DCOL:]
    z = jnp.maximum(jnp.concatenate([za, zb], axis=1), 0.0)
    h2 = jnp.dot(z, w2_ref[...],
                 preferred_element_type=jnp.float32,
                 precision=lax.Precision.HIGHEST)
    g2_ref[...] = h2 * dis


def _mid(s1, ga, gb, dis, b1, W2):
    return pl.pallas_call(
        _mid_body,
        grid=(N // BLK,),
        in_specs=[
            pl.BlockSpec((NC, 2, BLK, DCOL), lambda i: (0, 0, i, 0)),
            pl.BlockSpec((BLK, DCOL), lambda i: (i, 0)),
            pl.BlockSpec((BLK, DCOL), lambda i: (i, 0)),
            pl.BlockSpec((BLK, 1), lambda i: (i, 0)),
            pl.BlockSpec((1, D_HID), lambda i: (0, 0)),
            pl.BlockSpec((D_HID, D_OUT), lambda i: (0, 0)),
        ],
        out_specs=pl.BlockSpec((BLK, D_OUT), lambda i: (i, 0)),
        out_shape=jax.ShapeDtypeStruct((N, D_OUT), jnp.float32),
    )(s1, ga, gb, dis, b1, W2)


def _fin_body(s2_ref, g2_ref, dis_ref, b2_ref, out_ref):
    dis = dis_ref[...]
    out_ref[...] = (dis * (s2_ref[0] + s2_ref[1] + g2_ref[...])
                    + b2_ref[...])


def _fin(s2, g2, dis, b2):
    return pl.pallas_call(
        _fin_body,
        grid=(N // BLK,),
        in_specs=[
            pl.BlockSpec((NC, BLK, D_OUT), lambda i: (0, i, 0)),
            pl.BlockSpec((BLK, D_OUT), lambda i: (i, 0)),
            pl.BlockSpec((BLK, 1), lambda i: (i, 0)),
            pl.BlockSpec((1, D_OUT), lambda i: (0, 0)),
        ],
        out_specs=pl.BlockSpec((BLK, D_OUT), lambda i: (i, 0)),
        out_shape=jax.ShapeDtypeStruct((N, D_OUT), jnp.float32),
    )(s2, g2, dis, b2)


def kernel(x, edge_index, edge_weight, W1, b1, W2, b2):
    ei = edge_index.astype(jnp.int32)
    src = ei[0]
    dst = ei[1]
    ew = edge_weight.astype(jnp.float32)
    zeros_nd = jnp.zeros((N, DCOL), jnp.float32)

    degp0, degp1 = _deg_partials(dst, ew, jnp.zeros((NPAD,), jnp.float32))
    degp = jnp.stack([degp0[:N], degp1[:N]]).reshape(NC, N, 1)
    dis, g1a, g1b = _mm1(degp, x, W1)
    s1 = _spmm2_partials(g1a, g1b, src, dst, ew, zeros_nd)
    g2 = _mid(s1, g1a, g1b, dis, b1.reshape(1, D_HID), W2)
    s2 = _spmm_partials(g2, src, dst, ew, zeros_nd)
    return _fin(s2, g2, dis, b2.reshape(1, D_OUT))


# speedup vs baseline: 2.4062x; 1.0220x over previous
import functools

import jax
import jax.numpy as jnp
from jax import lax
from jax.experimental import pallas as pl
from jax.experimental.pallas import tpu as pltpu
from jax.experimental.pallas import tpu_sc as plsc

N = 10000
E = 320000
D_IN = 128
D_HID = 256
D_OUT = 128
DCOL = 128

NC = 2
NS = 16
NW = NC * NS
EPW = E // NW
C = 80
NCHUNK = EPW // C

DRAIN_ROWS = 624

_MESH = plsc.VectorSubcoreMesh(
    core_axis_name="c", subcore_axis_name="s", num_cores=NC, num_subcores=NS
)


NPAD = 10240


def _deg_body(dst_hbm, ew_hbm, zeros_hbm, out0_hbm, out1_hbm,
              acc_sh, dst_v0, ew_v0, lsem0, ssem0,
              dst_v1, ew_v1, lsem1, ssem1,
              dst_v2, ew_v2, lsem2, ssem2):
    c = lax.axis_index("c")
    s = lax.axis_index("s")
    wid = c * NS + s

    zrows = NPAD // NS
    pltpu.sync_copy(zeros_hbm.at[pl.ds(s * zrows, zrows)],
                    acc_sh.at[pl.ds(s * zrows, zrows)])

    plsc.subcore_barrier()

    base = wid * EPW
    bufs = ((dst_v0, ew_v0, lsem0, ssem0), (dst_v1, ew_v1, lsem1, ssem1),
            (dst_v2, ew_v2, lsem2, ssem2))

    def _wait_scatter(buf):
        dst_v, ew_v, _, ssem = buf
        pltpu.make_async_copy(ew_v, acc_sh.at[dst_v], ssem).wait()

    def _load(k, buf):
        dst_v, ew_v, lsem, _ = buf
        off = base + k * C
        pltpu.async_copy(dst_hbm.at[pl.ds(off, C)], dst_v, lsem)
        pltpu.async_copy(ew_hbm.at[pl.ds(off, C)], ew_v, lsem)

    def _wait_load(k, buf):
        dst_v, ew_v, lsem, _ = buf
        off = base + k * C
        pltpu.make_async_copy(dst_hbm.at[pl.ds(off, C)], dst_v, lsem).wait()
        pltpu.make_async_copy(ew_hbm.at[pl.ds(off, C)], ew_v, lsem).wait()

    def _step(k, cur, nxt):
        dst_v, ew_v, _, ssem = cur

        @pl.when(k + 2 < NCHUNK)
        def _():
            @pl.when(k >= 1)
            def _():
                _wait_scatter(nxt)

            _load(k + 2, nxt)

        _wait_load(k, cur)
        pltpu.async_copy(ew_v, acc_sh.at[dst_v], ssem, add=True)

    _load(0, bufs[0])
    _load(1, bufs[1])

    @pl.loop(0, NCHUNK)
    def _(k):
        for par in range(3):
            @pl.when(k % 3 == par)
            def _(par=par):
                _step(k, bufs[par], bufs[(par + 2) % 3])

    _wait_scatter(bufs[(NCHUNK - 3) % 3])
    _wait_scatter(bufs[(NCHUNK - 2) % 3])
    _wait_scatter(bufs[(NCHUNK - 1) % 3])

    plsc.subcore_barrier()

    @pl.when((s == 0) & (c == 0))
    def _():
        pltpu.sync_copy(acc_sh, out0_hbm)

    @pl.when((s == 0) & (c == 1))
    def _():
        pltpu.sync_copy(acc_sh, out1_hbm)


def _deg_partials(dst, ew, zeros_pad):
    return pl.kernel(
        _deg_body,
        out_type=(jax.ShapeDtypeStruct((NPAD,), jnp.float32),
                  jax.ShapeDtypeStruct((NPAD,), jnp.float32)),
        mesh=_MESH,
        scratch_types=[
            pltpu.VMEM_SHARED((NPAD,), jnp.float32),
            pltpu.VMEM((C,), jnp.int32),
            pltpu.VMEM((C,), jnp.float32),
            pltpu.SemaphoreType.DMA,
            pltpu.SemaphoreType.DMA,
            pltpu.VMEM((C,), jnp.int32),
            pltpu.VMEM((C,), jnp.float32),
            pltpu.SemaphoreType.DMA,
            pltpu.SemaphoreType.DMA,
            pltpu.VMEM((C,), jnp.int32),
            pltpu.VMEM((C,), jnp.float32),
            pltpu.SemaphoreType.DMA,
            pltpu.SemaphoreType.DMA,
        ],
    )(dst, ew, zeros_pad)


def _spmm_generic(g_hbms, drains, src_hbm, dst_hbm, ew_hbm, zeros_hbm,
                  acc_sh, src_big, bufs):
    c = lax.axis_index("c")
    s = lax.axis_index("s")
    wid = c * NS + s
    base = wid * EPW
    row0 = s * DRAIN_ROWS
    last = (NS - 1) * DRAIN_ROWS

    def _zero_acc():
        @pl.when(s < NS - 1)
        def _():
            pltpu.sync_copy(zeros_hbm.at[pl.ds(row0, DRAIN_ROWS)],
                            acc_sh.at[pl.ds(row0, DRAIN_ROWS)])

        @pl.when(s == NS - 1)
        def _():
            pltpu.sync_copy(zeros_hbm.at[pl.ds(last, N - last)],
                            acc_sh.at[pl.ds(last, N - last)])

    _zero_acc()
    pltpu.sync_copy(src_hbm.at[pl.ds(base, EPW)], src_big)
    plsc.subcore_barrier()

    def _wait_scatter(buf):
        dst_v, _, rows_v, _, _, ssem = buf
        pltpu.make_async_copy(rows_v, acc_sh.at[dst_v], ssem).wait()

    for h, g_hbm in enumerate(g_hbms):
        def _start_gather(k, buf, g_hbm=g_hbm):
            dst_v, ew_v, rows_v, lsem, gsem, _ = buf
            pltpu.async_copy(dst_hbm.at[pl.ds(base + k * C, C)], dst_v, lsem)
            pltpu.async_copy(ew_hbm.at[pl.ds(base + k * C, C)], ew_v, lsem)
            pltpu.async_copy(g_hbm.at[src_big.at[pl.ds(k * C, C)]],
                             rows_v, gsem)

        def _step(k, cur, nxt, g_hbm=g_hbm, _start_gather=_start_gather):
            dst_v, ew_v, rows_v, lsem, gsem, ssem = cur
            pltpu.make_async_copy(g_hbm.at[src_big.at[pl.ds(k * C, C)]],
                                  rows_v, gsem).wait()

            @pl.when(k + 2 < NCHUNK)
            def _():
                @pl.when(k >= 1)
                def _():
                    _wait_scatter(nxt)

                _start_gather(k + 2, nxt)

            pltpu.make_async_copy(ew_hbm.at[pl.ds(base + k * C, C)],
                                  ew_v, lsem).wait()

            @pl.loop(0, C // 16)
            def _(gidx):
                wvec = ew_v[pl.ds(gidx * 16, 16)]
                for lane in range(16):
                    w = jnp.full((16,), wvec[lane])
                    r = gidx * 16 + lane
                    for j in range(DCOL // 16):
                        sl = pl.ds(j * 16, 16)
                        rows_v[r, sl] = rows_v[r, sl] * w

            pltpu.make_async_copy(dst_hbm.at[pl.ds(base + k * C, C)],
                                  dst_v, lsem).wait()
            pltpu.async_copy(rows_v, acc_sh.at[dst_v], ssem, add=True)

        _start_gather(0, bufs[0])
        _start_gather(1, bufs[1])

        @pl.loop(0, NCHUNK)
        def _(k):
            for par in range(3):
                @pl.when(k % 3 == par)
                def _(par=par, _step=_step):
                    _step(k, bufs[par], bufs[(par + 2) % 3])

        _wait_scatter(bufs[(NCHUNK - 3) % 3])
        _wait_scatter(bufs[(NCHUNK - 2) % 3])
        _wait_scatter(bufs[(NCHUNK - 1) % 3])

        plsc.subcore_barrier()

        drain = drains[h]

        @pl.when(s < NS - 1)
        def _(drain=drain):
            pltpu.sync_copy(acc_sh.at[pl.ds(row0, DRAIN_ROWS)],
                            drain(pl.ds(row0, DRAIN_ROWS)))

        @pl.when(s == NS - 1)
        def _(drain=drain):
            pltpu.sync_copy(acc_sh.at[pl.ds(last, N - last)],
                            drain(pl.ds(last, N - last)))

        if h + 1 < len(g_hbms):
            _zero_acc()
            plsc.subcore_barrier()


def _spmm_body(g_hbm, src_hbm, dst_hbm, ew_hbm, zeros_hbm, out_hbm,
               acc_sh, src_big,
               dst_v0, ew_v0, rows_v0, lsem0, gsem0, ssem0,
               dst_v1, ew_v1, rows_v1, lsem1, gsem1, ssem1,
               dst_v2, ew_v2, rows_v2, lsem2, gsem2, ssem2):
    c = lax.axis_index("c")
    bufs = ((dst_v0, ew_v0, rows_v0, lsem0, gsem0, ssem0),
            (dst_v1, ew_v1, rows_v1, lsem1, gsem1, ssem1),
            (dst_v2, ew_v2, rows_v2, lsem2, gsem2, ssem2))
    _spmm_generic((g_hbm,), (lambda sl: out_hbm.at[c, sl],),
                  src_hbm, dst_hbm, ew_hbm, zeros_hbm, acc_sh, src_big, bufs)


def _spmm2_body(ga_hbm, gb_hbm, src_hbm, dst_hbm, ew_hbm, zeros_hbm, out_hbm,
                acc_sh, src_big,
                dst_v0, ew_v0, rows_v0, lsem0, gsem0, ssem0,
                dst_v1, ew_v1, rows_v1, lsem1, gsem1, ssem1,
                dst_v2, ew_v2, rows_v2, lsem2, gsem2, ssem2):
    c = lax.axis_index("c")
    bufs = ((dst_v0, ew_v0, rows_v0, lsem0, gsem0, ssem0),
            (dst_v1, ew_v1, rows_v1, lsem1, gsem1, ssem1),
            (dst_v2, ew_v2, rows_v2, lsem2, gsem2, ssem2))
    _spmm_generic((ga_hbm, gb_hbm),
                  (lambda sl: out_hbm.at[c, 0, sl],
                   lambda sl: out_hbm.at[c, 1, sl]),
                  src_hbm, dst_hbm, ew_hbm, zeros_hbm, acc_sh, src_big, bufs)


def _spmm_partials(g, src, dst, ew, zeros_nd):
    ring = []
    for _ in range(3):
        ring += [
            pltpu.VMEM((C,), jnp.int32),
            pltpu.VMEM((C,), jnp.float32),
            pltpu.VMEM((C, DCOL), jnp.float32),
            pltpu.SemaphoreType.DMA,
            pltpu.SemaphoreType.DMA,
            pltpu.SemaphoreType.DMA,
        ]
    return pl.kernel(
        _spmm_body,
        out_type=jax.ShapeDtypeStruct((NC, N, DCOL), jnp.float32),
        mesh=_MESH,
        scratch_types=[
            pltpu.VMEM_SHARED((N, DCOL), jnp.float32),
            pltpu.VMEM((EPW,), jnp.int32),
        ] + ring,
    )(g, src, dst, ew, zeros_nd)


def _spmm2_partials(ga, gb, src, dst, ew, zeros_nd):
    ring = []
    for _ in range(3):
        ring += [
            pltpu.VMEM((C,), jnp.int32),
            pltpu.VMEM((C,), jnp.float32),
            pltpu.VMEM((C, DCOL), jnp.float32),
            pltpu.SemaphoreType.DMA,
            pltpu.SemaphoreType.DMA,
            pltpu.SemaphoreType.DMA,
        ]
    return pl.kernel(
        _spmm2_body,
        out_type=jax.ShapeDtypeStruct((NC, 2, N, DCOL), jnp.float32),
        mesh=_MESH,
        scratch_types=[
            pltpu.VMEM_SHARED((N, DCOL), jnp.float32),
            pltpu.VMEM((EPW,), jnp.int32),
        ] + ring,
    )(ga, gb, src, dst, ew, zeros_nd)


BLK = 1000


def _mm1_body(degp_ref, x_ref, w_ref, dis_ref, ga_ref, gb_ref):
    deg = degp_ref[0] + degp_ref[1] + 1.0
    dis = lax.rsqrt(deg)
    h = jnp.dot(x_ref[...], w_ref[...],
                preferred_element_type=jnp.float32,
                precision=lax.Precision.HIGHEST)
    g = h * dis
    dis_ref[...] = dis
    ga_ref[...] = g[:, :DCOL]
    gb_ref[...] = g[:, DCOL:]


def _mm1(degp, x, W1):
    return pl.pallas_call(
        _mm1_body,
        grid=(N // BLK,),
        in_specs=[
            pl.BlockSpec((NC, BLK, 1), lambda i: (0, i, 0)),
            pl.BlockSpec((BLK, D_IN), lambda i: (i, 0)),
            pl.BlockSpec((D_IN, D_HID), lambda i: (0, 0)),
        ],
        out_specs=[
            pl.BlockSpec((BLK, 1), lambda i: (i, 0)),
            pl.BlockSpec((BLK, DCOL), lambda i: (i, 0)),
            pl.BlockSpec((BLK, DCOL), lambda i: (i, 0)),
        ],
        out_shape=[
            jax.ShapeDtypeStruct((N, 1), jnp.float32),
            jax.ShapeDtypeStruct((N, DCOL), jnp.float32),
            jax.ShapeDtypeStruct((N, DCOL), jnp.float32),
        ],
    )(degp, x, W1)


def _mid_body(s1_ref, ga_ref, gb_ref, dis_ref, b1_ref, w2_ref, g2_ref):
    dis = dis_ref[...]
    b1 = b1_ref[...]
    za = dis * (s1_ref[0, 0] + s1_ref[1, 0] + ga_ref[...]) + b1[:, :DCOL]
    zb = dis * (s1_ref[0, 1] + s1_ref[1, 1] + gb_ref[...]) + b1[:, DCOL:]
    z = jnp.maximum(jnp.concatenate([za, zb], axis=1), 0.0)
    h2 = jnp.dot(z, w2_ref[...],
                 preferred_element_type=jnp.float32,
                 precision=lax.Precision.HIGHEST)
    g2_ref[...] = h2 * dis


def _mid(s1, ga, gb, dis, b1, W2):
    return pl.pallas_call(
        _mid_body,
        grid=(N // BLK,),
        in_specs=[
            pl.BlockSpec((NC, 2, BLK, DCOL), lambda i: (0, 0, i, 0)),
            pl.BlockSpec((BLK, DCOL), lambda i: (i, 0)),
            pl.BlockSpec((BLK, DCOL), lambda i: (i, 0)),
            pl.BlockSpec((BLK, 1), lambda i: (i, 0)),
            pl.BlockSpec((1, D_HID), lambda i: (0, 0)),
            pl.BlockSpec((D_HID, D_OUT), lambda i: (0, 0)),
        ],
        out_specs=pl.BlockSpec((BLK, D_OUT), lambda i: (i, 0)),
        out_shape=jax.ShapeDtypeStruct((N, D_OUT), jnp.float32),
    )(s1, ga, gb, dis, b1, W2)


def _fin_body(s2_ref, g2_ref, dis_ref, b2_ref, out_ref):
    dis = dis_ref[...]
    out_ref[...] = (dis * (s2_ref[0] + s2_ref[1] + g2_ref[...])
                    + b2_ref[...])


def _fin(s2, g2, dis, b2):
    return pl.pallas_call(
        _fin_body,
        grid=(N // BLK,),
        in_specs=[
            pl.BlockSpec((NC, BLK, D_OUT), lambda i: (0, i, 0)),
            pl.BlockSpec((BLK, D_OUT), lambda i: (i, 0)),
            pl.BlockSpec((BLK, 1), lambda i: (i, 0)),
            pl.BlockSpec((1, D_OUT), lambda i: (0, 0)),
        ],
        out_specs=pl.BlockSpec((BLK, D_OUT), lambda i: (i, 0)),
        out_shape=jax.ShapeDtypeStruct((N, D_OUT), jnp.float32),
    )(s2, g2, dis, b2)


def kernel(x, edge_index, edge_weight, W1, b1, W2, b2):
    ei = edge_index.astype(jnp.int32)
    src = ei[0]
    dst = ei[1]
    ew = edge_weight.astype(jnp.float32)
    zeros_nd = jnp.zeros((N, DCOL), jnp.float32)

    degp0, degp1 = _deg_partials(dst, ew, jnp.zeros((NPAD,), jnp.float32))
    degp = jnp.stack([degp0[:N], degp1[:N]]).reshape(NC, N, 1)
    dis, g1a, g1b = _mm1(degp, x, W1)
    s1 = _spmm2_partials(g1a, g1b, src, dst, ew, zeros_nd)
    g2 = _mid(s1, g1a, g1b, dis, b1.reshape(1, D_HID), W2)
    s2 = _spmm_partials(g2, src, dst, ew, zeros_nd)
    return _fin(s2, g2, dis, b2.reshape(1, D_OUT))


# speedup vs baseline: 2.4090x; 1.0012x over previous
import jax
import jax.numpy as jnp
from jax import lax
from jax.experimental import pallas as pl
from jax.experimental.pallas import tpu as pltpu
from jax.experimental.pallas import tpu_sc as plsc

N = 10000
E = 320000
D_IN = 128
D_HID = 256
D_OUT = 128
DCOL = 128

NC = 2
NS = 16
NW = NC * NS
EPW = E // NW
C = 80
NCHUNK = EPW // C

DRAIN_ROWS = 624

_MESH = plsc.VectorSubcoreMesh(
    core_axis_name="c", subcore_axis_name="s", num_cores=NC, num_subcores=NS
)


NPAD = 10240


def _deg_body(dst_hbm, ew_hbm, zeros_hbm, out0_hbm, out1_hbm,
              acc_sh, dst_v0, ew_v0, lsem0, ssem0,
              dst_v1, ew_v1, lsem1, ssem1,
              dst_v2, ew_v2, lsem2, ssem2):
    c = lax.axis_index("c")
    s = lax.axis_index("s")
    wid = c * NS + s

    zrows = NPAD // NS
    pltpu.sync_copy(zeros_hbm.at[pl.ds(s * zrows, zrows)],
                    acc_sh.at[pl.ds(s * zrows, zrows)])

    plsc.subcore_barrier()

    base = wid * EPW
    bufs = ((dst_v0, ew_v0, lsem0, ssem0), (dst_v1, ew_v1, lsem1, ssem1),
            (dst_v2, ew_v2, lsem2, ssem2))

    def _wait_scatter(buf):
        dst_v, ew_v, _, ssem = buf
        pltpu.make_async_copy(ew_v, acc_sh.at[dst_v], ssem).wait()

    def _load(k, buf):
        dst_v, ew_v, lsem, _ = buf
        off = base + k * C
        pltpu.async_copy(dst_hbm.at[pl.ds(off, C)], dst_v, lsem)
        pltpu.async_copy(ew_hbm.at[pl.ds(off, C)], ew_v, lsem)

    def _wait_load(k, buf):
        dst_v, ew_v, lsem, _ = buf
        off = base + k * C
        pltpu.make_async_copy(dst_hbm.at[pl.ds(off, C)], dst_v, lsem).wait()
        pltpu.make_async_copy(ew_hbm.at[pl.ds(off, C)], ew_v, lsem).wait()

    def _step(k, cur, nxt):
        dst_v, ew_v, _, ssem = cur

        @pl.when(k + 2 < NCHUNK)
        def _():
            @pl.when(k >= 1)
            def _():
                _wait_scatter(nxt)

            _load(k + 2, nxt)

        _wait_load(k, cur)
        pltpu.async_copy(ew_v, acc_sh.at[dst_v], ssem, add=True)

    _load(0, bufs[0])
    _load(1, bufs[1])

    @pl.loop(0, NCHUNK)
    def _(k):
        for par in range(3):
            @pl.when(k % 3 == par)
            def _(par=par):
                _step(k, bufs[par], bufs[(par + 2) % 3])

    _wait_scatter(bufs[(NCHUNK - 3) % 3])
    _wait_scatter(bufs[(NCHUNK - 2) % 3])
    _wait_scatter(bufs[(NCHUNK - 1) % 3])

    plsc.subcore_barrier()

    @pl.when((s == 0) & (c == 0))
    def _():
        pltpu.sync_copy(acc_sh, out0_hbm)

    @pl.when((s == 0) & (c == 1))
    def _():
        pltpu.sync_copy(acc_sh, out1_hbm)


def _deg_partials(dst, ew, zeros_pad):
    return pl.kernel(
        _deg_body,
        out_type=(jax.ShapeDtypeStruct((NPAD,), jnp.float32),
                  jax.ShapeDtypeStruct((NPAD,), jnp.float32)),
        mesh=_MESH,
        scratch_types=[
            pltpu.VMEM_SHARED((NPAD,), jnp.float32),
            pltpu.VMEM((C,), jnp.int32),
            pltpu.VMEM((C,), jnp.float32),
            pltpu.SemaphoreType.DMA,
            pltpu.SemaphoreType.DMA,
            pltpu.VMEM((C,), jnp.int32),
            pltpu.VMEM((C,), jnp.float32),
            pltpu.SemaphoreType.DMA,
            pltpu.SemaphoreType.DMA,
            pltpu.VMEM((C,), jnp.int32),
            pltpu.VMEM((C,), jnp.float32),
            pltpu.SemaphoreType.DMA,
            pltpu.SemaphoreType.DMA,
        ],
    )(dst, ew, zeros_pad)


def _spmm_generic(g_hbms, drains, src_hbm, dst_hbm, ew_hbm, zeros_hbm,
                  acc_sh, src_big, bufs):
    c = lax.axis_index("c")
    s = lax.axis_index("s")
    wid = c * NS + s
    base = wid * EPW
    row0 = s * DRAIN_ROWS
    last = (NS - 1) * DRAIN_ROWS

    def _zero_acc():
        @pl.when(s < NS - 1)
        def _():
            pltpu.sync_copy(zeros_hbm.at[pl.ds(row0, DRAIN_ROWS)],
                            acc_sh.at[pl.ds(row0, DRAIN_ROWS)])

        @pl.when(s == NS - 1)
        def _():
            pltpu.sync_copy(zeros_hbm.at[pl.ds(last, N - last)],
                            acc_sh.at[pl.ds(last, N - last)])

    _zero_acc()
    pltpu.sync_copy(src_hbm.at[pl.ds(base, EPW)], src_big)
    plsc.subcore_barrier()

    def _wait_scatter(buf):
        dst_v, _, rows_v, _, _, ssem = buf
        pltpu.make_async_copy(rows_v, acc_sh.at[dst_v], ssem).wait()

    for h, g_hbm in enumerate(g_hbms):
        def _start_gather(k, buf, g_hbm=g_hbm):
            dst_v, ew_v, rows_v, lsem, gsem, _ = buf
            pltpu.async_copy(dst_hbm.at[pl.ds(base + k * C, C)], dst_v, lsem)
            pltpu.async_copy(ew_hbm.at[pl.ds(base + k * C, C)], ew_v, lsem)
            pltpu.async_copy(g_hbm.at[src_big.at[pl.ds(k * C, C)]],
                             rows_v, gsem)

        def _step(k, cur, nxt, g_hbm=g_hbm, _start_gather=_start_gather):
            dst_v, ew_v, rows_v, lsem, gsem, ssem = cur
            pltpu.make_async_copy(g_hbm.at[src_big.at[pl.ds(k * C, C)]],
                                  rows_v, gsem).wait()

            @pl.when(k + 2 < NCHUNK)
            def _():
                @pl.when(k >= 1)
                def _():
                    _wait_scatter(nxt)

                _start_gather(k + 2, nxt)

            pltpu.make_async_copy(ew_hbm.at[pl.ds(base + k * C, C)],
                                  ew_v, lsem).wait()

            @pl.loop(0, C // 16)
            def _(gidx):
                wvec = ew_v[pl.ds(gidx * 16, 16)]
                for lane in range(16):
                    w = jnp.full((16,), wvec[lane])
                    r = gidx * 16 + lane
                    for j in range(DCOL // 16):
                        sl = pl.ds(j * 16, 16)
                        rows_v[r, sl] = rows_v[r, sl] * w

            pltpu.make_async_copy(dst_hbm.at[pl.ds(base + k * C, C)],
                                  dst_v, lsem).wait()
            pltpu.async_copy(rows_v, acc_sh.at[dst_v], ssem, add=True)

        _start_gather(0, bufs[0])
        _start_gather(1, bufs[1])

        @pl.loop(0, NCHUNK)
        def _(k):
            for par in range(3):
                @pl.when(k % 3 == par)
                def _(par=par, _step=_step):
                    _step(k, bufs[par], bufs[(par + 2) % 3])

        _wait_scatter(bufs[(NCHUNK - 3) % 3])
        _wait_scatter(bufs[(NCHUNK - 2) % 3])
        _wait_scatter(bufs[(NCHUNK - 1) % 3])

        plsc.subcore_barrier()

        drain = drains[h]

        @pl.when(s < NS - 1)
        def _(drain=drain):
            pltpu.sync_copy(acc_sh.at[pl.ds(row0, DRAIN_ROWS)],
                            drain(pl.ds(row0, DRAIN_ROWS)))

        @pl.when(s == NS - 1)
        def _(drain=drain):
            pltpu.sync_copy(acc_sh.at[pl.ds(last, N - last)],
                            drain(pl.ds(last, N - last)))

        if h + 1 < len(g_hbms):
            _zero_acc()
            plsc.subcore_barrier()


def _spmm_body(g_hbm, src_hbm, dst_hbm, ew_hbm, zeros_hbm, out_hbm,
               acc_sh, src_big,
               dst_v0, ew_v0, rows_v0, lsem0, gsem0, ssem0,
               dst_v1, ew_v1, rows_v1, lsem1, gsem1, ssem1,
               dst_v2, ew_v2, rows_v2, lsem2, gsem2, ssem2):
    c = lax.axis_index("c")
    bufs = ((dst_v0, ew_v0, rows_v0, lsem0, gsem0, ssem0),
            (dst_v1, ew_v1, rows_v1, lsem1, gsem1, ssem1),
            (dst_v2, ew_v2, rows_v2, lsem2, gsem2, ssem2))
    _spmm_generic((g_hbm,), (lambda sl: out_hbm.at[c, sl],),
                  src_hbm, dst_hbm, ew_hbm, zeros_hbm, acc_sh, src_big, bufs)


def _spmm2_body(ga_hbm, gb_hbm, src_hbm, dst_hbm, ew_hbm, zeros_hbm, out_hbm,
                acc_sh, src_big,
                dst_v0, ew_v0, rows_v0, lsem0, gsem0, ssem0,
                dst_v1, ew_v1, rows_v1, lsem1, gsem1, ssem1,
                dst_v2, ew_v2, rows_v2, lsem2, gsem2, ssem2):
    c = lax.axis_index("c")
    bufs = ((dst_v0, ew_v0, rows_v0, lsem0, gsem0, ssem0),
            (dst_v1, ew_v1, rows_v1, lsem1, gsem1, ssem1),
            (dst_v2, ew_v2, rows_v2, lsem2, gsem2, ssem2))
    _spmm_generic((ga_hbm, gb_hbm),
                  (lambda sl: out_hbm.at[c, 0, sl],
                   lambda sl: out_hbm.at[c, 1, sl]),
                  src_hbm, dst_hbm, ew_hbm, zeros_hbm, acc_sh, src_big, bufs)


def _spmm_partials(g, src, dst, ew, zeros_nd):
    ring = []
    for _ in range(3):
        ring += [
            pltpu.VMEM((C,), jnp.int32),
            pltpu.VMEM((C,), jnp.float32),
            pltpu.VMEM((C, DCOL), jnp.float32),
            pltpu.SemaphoreType.DMA,
            pltpu.SemaphoreType.DMA,
            pltpu.SemaphoreType.DMA,
        ]
    return pl.kernel(
        _spmm_body,
        out_type=jax.ShapeDtypeStruct((NC, N, DCOL), jnp.float32),
        mesh=_MESH,
        scratch_types=[
            pltpu.VMEM_SHARED((N, DCOL), jnp.float32),
            pltpu.VMEM((EPW,), jnp.int32),
        ] + ring,
    )(g, src, dst, ew, zeros_nd)


def _spmm2_partials(ga, gb, src, dst, ew, zeros_nd):
    ring = []
    for _ in range(3):
        ring += [
            pltpu.VMEM((C,), jnp.int32),
            pltpu.VMEM((C,), jnp.float32),
            pltpu.VMEM((C, DCOL), jnp.float32),
            pltpu.SemaphoreType.DMA,
            pltpu.SemaphoreType.DMA,
            pltpu.SemaphoreType.DMA,
        ]
    return pl.kernel(
        _spmm2_body,
        out_type=jax.ShapeDtypeStruct((NC, 2, N, DCOL), jnp.float32),
        mesh=_MESH,
        scratch_types=[
            pltpu.VMEM_SHARED((N, DCOL), jnp.float32),
            pltpu.VMEM((EPW,), jnp.int32),
        ] + ring,
    )(ga, gb, src, dst, ew, zeros_nd)


BLK = 1000


def _mm1_body(degp_ref, x_ref, w_ref, dis_ref, ga_ref, gb_ref):
    deg = degp_ref[0] + degp_ref[1] + 1.0
    dis = lax.rsqrt(deg)
    h = jnp.dot(x_ref[...], w_ref[...],
                preferred_element_type=jnp.float32,
                precision=lax.Precision.HIGHEST)
    g = h * dis
    dis_ref[...] = dis
    ga_ref[...] = g[:, :DCOL]
    gb_ref[...] = g[:, DCOL:]


def _mm1(degp, x, W1):
    return pl.pallas_call(
        _mm1_body,
        grid=(N // BLK,),
        in_specs=[
            pl.BlockSpec((NC, BLK, 1), lambda i: (0, i, 0)),
            pl.BlockSpec((BLK, D_IN), lambda i: (i, 0)),
            pl.BlockSpec((D_IN, D_HID), lambda i: (0, 0)),
        ],
        out_specs=[
            pl.BlockSpec((BLK, 1), lambda i: (i, 0)),
            pl.BlockSpec((BLK, DCOL), lambda i: (i, 0)),
            pl.BlockSpec((BLK, DCOL), lambda i: (i, 0)),
        ],
        out_shape=[
            jax.ShapeDtypeStruct((N, 1), jnp.float32),
            jax.ShapeDtypeStruct((N, DCOL), jnp.float32),
            jax.ShapeDtypeStruct((N, DCOL), jnp.float32),
        ],
    )(degp, x, W1)


def _mid_body(s1_ref, ga_ref, gb_ref, dis_ref, b1_ref, w2_ref, g2_ref):
    dis = dis_ref[...]
    b1 = b1_ref[...]
    za = dis * (s1_ref[0, 0] + s1_ref[1, 0] + ga_ref[...]) + b1[:, :DCOL]
    zb = dis * (s1_ref[0, 1] + s1_ref[1, 1] + gb_ref[...]) + b1[:, DCOL:]
    z = jnp.maximum(jnp.concatenate([za, zb], axis=1), 0.0)
    h2 = jnp.dot(z, w2_ref[...],
                 preferred_element_type=jnp.float32,
                 precision=lax.Precision.HIGHEST)
    g2_ref[...] = h2 * dis


def _mid(s1, ga, gb, dis, b1, W2):
    return pl.pallas_call(
        _mid_body,
        grid=(N // BLK,),
        in_specs=[
            pl.BlockSpec((NC, 2, BLK, DCOL), lambda i: (0, 0, i, 0)),
            pl.BlockSpec((BLK, DCOL), lambda i: (i, 0)),
            pl.BlockSpec((BLK, DCOL), lambda i: (i, 0)),
            pl.BlockSpec((BLK, 1), lambda i: (i, 0)),
            pl.BlockSpec((1, D_HID), lambda i: (0, 0)),
            pl.BlockSpec((D_HID, D_OUT), lambda i: (0, 0)),
        ],
        out_specs=pl.BlockSpec((BLK, D_OUT), lambda i: (i, 0)),
        out_shape=jax.ShapeDtypeStruct((N, D_OUT), jnp.float32),
    )(s1, ga, gb, dis, b1, W2)


def _fin_body(s2_ref, g2_ref, dis_ref, b2_ref, out_ref):
    dis = dis_ref[...]
    out_ref[...] = (dis * (s2_ref[0] + s2_ref[1] + g2_ref[...])
                    + b2_ref[...])


def _fin(s2, g2, dis, b2):
    return pl.pallas_call(
        _fin_body,
        grid=(N // BLK,),
        in_specs=[
            pl.BlockSpec((NC, BLK, D_OUT), lambda i: (0, i, 0)),
            pl.BlockSpec((BLK, D_OUT), lambda i: (i, 0)),
            pl.BlockSpec((BLK, 1), lambda i: (i, 0)),
            pl.BlockSpec((1, D_OUT), lambda i: (0, 0)),
        ],
        out_specs=pl.BlockSpec((BLK, D_OUT), lambda i: (i, 0)),
        out_shape=jax.ShapeDtypeStruct((N, D_OUT), jnp.float32),
    )(s2, g2, dis, b2)


def kernel(x, edge_index, edge_weight, W1, b1, W2, b2):
    ei = edge_index.astype(jnp.int32)
    src = ei[0]
    dst = ei[1]
    ew = edge_weight.astype(jnp.float32)
    zeros_nd = jnp.zeros((N, DCOL), jnp.float32)

    degp0, degp1 = _deg_partials(dst, ew, jnp.zeros((NPAD,), jnp.float32))
    degp = jnp.stack([degp0[:N], degp1[:N]]).reshape(NC, N, 1)
    dis, g1a, g1b = _mm1(degp, x, W1)
    s1 = _spmm2_partials(g1a, g1b, src, dst, ew, zeros_nd)
    g2 = _mid(s1, g1a, g1b, dis, b1.reshape(1, D_HID), W2)
    s2 = _spmm_partials(g2, src, dst, ew, zeros_nd)
    return _fin(s2, g2, dis, b2.reshape(1, D_OUT))
